# baseline (speedup 1.0000x reference)
import functools

import jax
import jax.numpy as jnp
from jax import lax
from jax.experimental import pallas as pl
from jax.experimental.pallas import tpu as pltpu
from jax.experimental.pallas import tpu_sc as plsc

NEG_BIG = -3.0e38
_SC_PARAMS = pltpu.CompilerParams(use_tc_tiling_on_sc=False,
                                  needs_layout_passes=False)
CH = 128
NW = 32
NSUB = 16


def _tree_sum(vs):
    vs = list(vs)
    while len(vs) > 1:
        nxt = [vs[i] + vs[i + 1] for i in range(0, len(vs) - 1, 2)]
        if len(vs) % 2:
            nxt.append(vs[-1])
        vs = nxt
    return vs[0]


def _build_proj(NP, D, H):
    TN = 512
    grid = (NP // TN,)

    def body(x_ref, w_ref, b_ref, q_ref, k_ref, v0_ref, v1_ref, s_ref):
        res = jnp.dot(x_ref[...], w_ref[...],
                      preferred_element_type=jnp.float32) + b_ref[...]
        q_ref[...] = res[:, 0:H]
        k_ref[...] = res[:, H:2 * H]
        v0_ref[...] = res[:, 2 * H:2 * H + H // 2]
        v1_ref[...] = res[:, 2 * H + H // 2:3 * H]
        s_ref[...] = res[:, 3 * H:4 * H]

    return pl.pallas_call(
        body,
        grid=grid,
        in_specs=[
            pl.BlockSpec((TN, D), lambda i: (i, 0)),
            pl.BlockSpec((D, 4 * H), lambda i: (0, 0)),
            pl.BlockSpec((1, 4 * H), lambda i: (0, 0)),
        ],
        out_specs=[
            pl.BlockSpec((TN, H), lambda i: (i, 0)),
            pl.BlockSpec((TN, H), lambda i: (i, 0)),
            pl.BlockSpec((TN, H // 2), lambda i: (i, 0)),
            pl.BlockSpec((TN, H // 2), lambda i: (i, 0)),
            pl.BlockSpec((TN, H), lambda i: (i, 0)),
        ],
        out_shape=[
            jax.ShapeDtypeStruct((NP, H), jnp.float32),
            jax.ShapeDtypeStruct((NP, H), jnp.float32),
            jax.ShapeDtypeStruct((NP, H // 2), jnp.float32),
            jax.ShapeDtypeStruct((NP, H // 2), jnp.float32),
            jax.ShapeDtypeStruct((NP, H), jnp.float32),
        ],
    )


def _build_k1(NP, Ep, H, CH1, CHW1, EPW):
    inv_sqrt_h = 1.0 / (H ** 0.5)
    mesh = plsc.VectorSubcoreMesh(core_axis_name="c", subcore_axis_name="s")

    @functools.partial(
        pl.kernel,
        out_type=(jax.ShapeDtypeStruct((NW, EPW), jnp.float32),
                  jax.ShapeDtypeStruct((NW, 16), jnp.float32)),
        mesh=mesh,
        compiler_params=_SC_PARAMS,
        scratch_types=[
            pltpu.VMEM((CHW1, CH1), jnp.int32),
            pltpu.VMEM((CHW1, CH1), jnp.int32),
            pltpu.VMEM((CH1, H), jnp.float32),
            pltpu.VMEM((CH1, H), jnp.float32),
            pltpu.VMEM((CH1, H), jnp.float32),
            pltpu.VMEM((CH1, H), jnp.float32),
            pltpu.VMEM((EPW,), jnp.float32),
            pltpu.VMEM((16, 16), jnp.float32),
            pltpu.VMEM((16,), jnp.float32),
            pltpu.SemaphoreType.DMA,
            pltpu.SemaphoreType.DMA,
            pltpu.SemaphoreType.DMA,
            pltpu.SemaphoreType.DMA,
        ],
    )
    def k1(q_hbm, k_hbm, dst_hbm, src_hbm, alpha_hbm, mx_hbm,
           dst2d, src2d, qr0, kr0, qr1, kr1, alphabig, tbuf, mxbuf,
           sq0, sk0, sq1, sk1):
        c = lax.axis_index("c")
        s = lax.axis_index("s")
        wid = s * 2 + c
        pltpu.sync_copy(dst_hbm.at[wid], dst2d)
        pltpu.sync_copy(src_hbm.at[wid], src2d)
        iota = jnp.arange(16, dtype=jnp.int32)
        inv = jnp.float32(inv_sqrt_h)
        bufs = ((qr0, kr0, sq0, sk0), (qr1, kr1, sq1, sk1))

        def issue(t, b):
            qr, kr, sq, sk = bufs[b]
            pltpu.async_copy(q_hbm.at[dst2d.at[t]], qr, sq)
            pltpu.async_copy(k_hbm.at[src2d.at[t]], kr, sk)

        def wait(t, b):
            qr, kr, sq, sk = bufs[b]
            pltpu.make_async_copy(q_hbm.at[dst2d.at[t]], qr, sq).wait()
            pltpu.make_async_copy(k_hbm.at[src2d.at[t]], kr, sk).wait()

        def compute(t, b, mxv):
            qr, kr, _, _ = bufs[b]
            for grp in range(CH1 // 16):

                def ebody(e, carry):
                    r = grp * 16 + e
                    ps = [qr[r, pl.ds(j * 16, 16)] * kr[r, pl.ds(j * 16, 16)]
                          for j in range(H // 16)]
                    tbuf[e, pl.ds(0, 16)] = _tree_sum(ps)
                    return carry

                lax.fori_loop(0, 16, ebody, jnp.int32(0))
                cols = [plsc.load_gather(tbuf, [iota, jnp.full((16,), j, jnp.int32)])
                        for j in range(16)]
                a16 = _tree_sum(cols) * inv
                alphabig[pl.ds(t * CH1 + grp * 16, 16)] = a16
                mxv = jnp.maximum(mxv, a16)
            return mxv

        issue(0, 0)

        def pair_body(p, mxv):
            t0 = p * 2
            t1 = t0 + 1
            issue(t1, 1)
            wait(t0, 0)
            mxv = compute(t0, 0, mxv)

            @pl.when(t0 + 2 < CHW1)
            def _():
                issue(t0 + 2, 0)

            wait(t1, 1)
            mxv = compute(t1, 1, mxv)
            return mxv

        mxv = lax.fori_loop(0, CHW1 // 2, pair_body,
                            jnp.full((16,), NEG_BIG, jnp.float32))
        mxbuf[...] = mxv
        pltpu.sync_copy(alphabig, alpha_hbm.at[wid])
        pltpu.sync_copy(mxbuf, mx_hbm.at[wid])

    return k1


def _build_k2(ND, SLICE, CHW, EPW):
    mesh = plsc.VectorSubcoreMesh(core_axis_name="c", subcore_axis_name="s")

    @functools.partial(
        pl.kernel,
        out_type=(jax.ShapeDtypeStruct((NW, EPW), jnp.float32),
                  jax.ShapeDtypeStruct((2, ND), jnp.float32)),
        mesh=mesh,
        compiler_params=_SC_PARAMS,
        scratch_types=[
            pltpu.VMEM((NW, 16), jnp.float32),
            pltpu.VMEM((CHW, CH), jnp.int32),
            pltpu.VMEM((EPW,), jnp.float32),
            pltpu.VMEM((EPW,), jnp.float32),
            pltpu.VMEM_SHARED((ND,), jnp.float32),
        ],
    )
    def k2(alpha_hbm, dst_hbm, mx_hbm, znd_hbm, ex_hbm, den_hbm,
           mxbuf, dst2d, alphabig, exbig, denom_sp):
        c = lax.axis_index("c")
        s = lax.axis_index("s")
        wid = s * 2 + c
        pltpu.sync_copy(mx_hbm, mxbuf)

        def mbody(i, m):
            return jnp.maximum(m, mxbuf[i])

        m = lax.fori_loop(0, NW, mbody, jnp.full((16,), NEG_BIG, jnp.float32))
        cmax = jnp.max(m)
        cvec = jnp.full((16,), cmax)
        pltpu.sync_copy(znd_hbm.at[pl.ds(s * SLICE, SLICE)],
                        denom_sp.at[pl.ds(s * SLICE, SLICE)])
        pltpu.sync_copy(alpha_hbm.at[wid], alphabig)
        pltpu.sync_copy(dst_hbm.at[wid], dst2d)

        def gbody(g, carry):
            sl = pl.ds(g * 16, 16)
            exbig[sl] = jnp.exp(alphabig[sl] - cvec)
            return carry

        lax.fori_loop(0, EPW // 16, gbody, jnp.int32(0))
        pltpu.sync_copy(exbig, ex_hbm.at[wid])
        plsc.subcore_barrier()

        def sbody(t, carry):
            pltpu.sync_copy(exbig.at[pl.ds(t * CH, CH)],
                            denom_sp.at[dst2d.at[t]], add=True)
            return carry

        lax.fori_loop(0, CHW, sbody, jnp.int32(0))
        plsc.subcore_barrier()
        pltpu.sync_copy(denom_sp.at[pl.ds(s * SLICE, SLICE)],
                        den_hbm.at[c, pl.ds(s * SLICE, SLICE)])

    return k2


def _build_k3(N, NP, ND, SLICE, HH, CHS2, CH2, ESUB):
    last_rows = N - (NSUB - 1) * SLICE
    GC = 16
    NG = CHS2 // GC
    DB = ND // 4
    mesh = plsc.VectorSubcoreMesh(core_axis_name="c", subcore_axis_name="s")

    @functools.partial(
        pl.kernel,
        out_type=jax.ShapeDtypeStruct((2, N, HH), jnp.float32),
        mesh=mesh,
        compiler_params=_SC_PARAMS,
        scratch_types=[
            pltpu.VMEM((ND,), jnp.float32),
            pltpu.VMEM((DB,), jnp.float32),
            pltpu.VMEM((GC, CH2), jnp.int32),
            pltpu.VMEM((GC, CH2), jnp.int32),
            pltpu.VMEM((GC * CH2,), jnp.float32),
            pltpu.VMEM((CH2,), jnp.float32),
            pltpu.VMEM((CH2, HH), jnp.float32),
            pltpu.VMEM((CH2, HH), jnp.float32),
            pltpu.VMEM_SHARED((ND, HH), jnp.float32),
            pltpu.SemaphoreType.DMA,
            pltpu.SemaphoreType.DMA,
            pltpu.SemaphoreType.DMA,
            pltpu.SemaphoreType.DMA,
        ],
    )
    def k3(vcat_hbm, ex_hbm, dst_hbm, src_hbm, den_hbm, zagg_hbm, agg_hbm,
           rdenom, dbuf, dstg, srcg, exg, wbuf, vr0, vr1, agg_sp,
           sg0, sg1, ss0, ss1):
        c = lax.axis_index("c")
        s = lax.axis_index("s")
        srcoff = c * NP
        pltpu.sync_copy(den_hbm.at[0], rdenom)
        for blk in range(4):
            pltpu.sync_copy(den_hbm.at[1, pl.ds(blk * DB, DB)], dbuf)

            def rbody(i, carry, _blk=blk):
                sl16 = pl.ds(_blk * DB + i * 16, 16)
                rdenom[sl16] = 1.0 / (rdenom[sl16] + dbuf[pl.ds(i * 16, 16)]
                                      + jnp.float32(1e-16))
                return carry

            lax.fori_loop(0, DB // 16, rbody, jnp.int32(0))
        pltpu.sync_copy(zagg_hbm, agg_sp.at[pl.ds(s * SLICE, SLICE)])
        plsc.subcore_barrier()
        bufs = ((vr0, sg0, ss0), (vr1, sg1, ss1))

        def issue_g(t, b):
            vr, sg, _ = bufs[b]
            pltpu.async_copy(vcat_hbm.at[srcg.at[t]], vr, sg)

        def wait_g(t, b):
            vr, sg, _ = bufs[b]
            pltpu.make_async_copy(vcat_hbm.at[srcg.at[t]], vr, sg).wait()

        def issue_sct(t, b):
            vr, _, ss = bufs[b]
            pltpu.async_copy(vr, agg_sp.at[dstg.at[t]], ss, add=True)

        def wait_sct(b):
            vr, _, ss = bufs[b]
            pltpu.make_async_copy(zagg_hbm.at[pl.ds(0, CH2)], vr, ss).wait()

        def scale(t, b):
            vr, _, _ = bufs[b]
            for grp in range(CH2 // 16):
                sl = pl.ds(grp * 16, 16)
                d16 = dstg[t, sl]
                rd = plsc.load_gather(rdenom, [d16])
                wbuf[sl] = exg[pl.ds(t * CH2 + grp * 16, 16)] * rd

            def ebody(e, carry3):
                wsp = plsc.load_gather(wbuf, [jnp.full((16,), e, jnp.int32)])
                for cb in range(HH // 16):
                    slc = pl.ds(cb * 16, 16)
                    vr[e, slc] = vr[e, slc] * wsp
                return carry3

            lax.fori_loop(0, CH2, ebody, jnp.int32(0), unroll=2)

        def group_body(gi, carry):
            pltpu.sync_copy(dst_hbm.at[s, pl.ds(gi * GC, GC)], dstg)
            pltpu.sync_copy(src_hbm.at[s, pl.ds(gi * GC, GC)], srcg)
            pltpu.sync_copy(ex_hbm.at[s, pl.ds(gi * GC * CH2, GC * CH2)], exg)

            def offbody(t, carry2):
                for j in range(CH2 // 16):
                    sl = pl.ds(j * 16, 16)
                    srcg[t, sl] = srcg[t, sl] + srcoff
                return carry2

            lax.fori_loop(0, GC, offbody, jnp.int32(0))
            issue_g(0, 0)
            issue_g(1, 1)

            def pair_body(p, carry2):
                t0 = p * 2
                t1 = t0 + 1
                wait_g(t0, 0)
                scale(t0, 0)
                issue_sct(t0, 0)
                wait_g(t1, 1)
                scale(t1, 1)
                issue_sct(t1, 1)

                @pl.when(t0 + 2 < GC)
                def _():
                    wait_sct(0)
                    issue_g(t0 + 2, 0)

                @pl.when(t1 + 2 < GC)
                def _():
                    wait_sct(1)
                    issue_g(t1 + 2, 1)

                return carry2

            lax.fori_loop(0, GC // 2, pair_body, jnp.int32(0))
            wait_sct(0)
            wait_sct(1)
            return carry

        lax.fori_loop(0, NG, group_body, jnp.int32(0))
        plsc.subcore_barrier()

        @pl.when(s != NSUB - 1)
        def _():
            pltpu.sync_copy(agg_sp.at[pl.ds(s * SLICE, SLICE)],
                            agg_hbm.at[c, pl.ds(s * SLICE, SLICE)])

        @pl.when(s == NSUB - 1)
        def _():
            pltpu.sync_copy(
                agg_sp.at[pl.ds((NSUB - 1) * SLICE, last_rows)],
                agg_hbm.at[c, pl.ds((NSUB - 1) * SLICE, last_rows)])

    return k3


def _build_pool(N, H):
    TN = 400
    grid = (N // TN,)

    def body(a0_ref, a1_ref, sx_ref, out_ref):
        i = pl.program_id(0)

        @pl.when(i == 0)
        def _():
            out_ref[...] = jnp.full_like(out_ref, NEG_BIG)

        h = jnp.concatenate([a0_ref[...], a1_ref[...]], axis=1) + sx_ref[...]
        m = jnp.max(h, axis=0, keepdims=True)
        out_ref[...] = jnp.maximum(out_ref[...], jnp.broadcast_to(m, out_ref.shape))

    return pl.pallas_call(
        body,
        grid=grid,
        in_specs=[
            pl.BlockSpec((TN, H // 2), lambda i: (i, 0)),
            pl.BlockSpec((TN, H // 2), lambda i: (i, 0)),
            pl.BlockSpec((TN, H), lambda i: (i, 0)),
        ],
        out_specs=pl.BlockSpec((8, H), lambda i: (0, 0)),
        out_shape=jax.ShapeDtypeStruct((8, H), jnp.float32),
    )


def _build_mlp1(B, Gp, H, P, M):
    TK = 1024
    nk = Gp // TK
    grid = (nk,)

    def body(ctrl_ref, w1a_ref, pert_ref, wp_ref, bp_ref, w1b_ref, w1c_ref,
             pooled_ref, bm1_ref, out_ref):
        i = pl.program_id(0)

        @pl.when(i == 0)
        def _():
            out_ref[...] = jnp.zeros_like(out_ref)

        out_ref[...] += jnp.dot(ctrl_ref[...], w1a_ref[...],
                                preferred_element_type=jnp.float32)

        @pl.when(i == nk - 1)
        def _():
            emb = jnp.dot(pert_ref[...], wp_ref[...],
                          preferred_element_type=jnp.float32) + bp_ref[...]
            acc2 = jnp.dot(emb, w1b_ref[...], preferred_element_type=jnp.float32)
            t = jnp.dot(pooled_ref[0:1, :], w1c_ref[...],
                        preferred_element_type=jnp.float32)
            z = out_ref[...] + acc2 + t + bm1_ref[...]
            out_ref[...] = jax.nn.softplus(z)

    return pl.pallas_call(
        body,
        grid=grid,
        in_specs=[
            pl.BlockSpec((B, TK), lambda i: (0, i)),
            pl.BlockSpec((TK, M), lambda i: (i, 0)),
            pl.BlockSpec((B, P), lambda i: (0, 0)),
            pl.BlockSpec((P, P), lambda i: (0, 0)),
            pl.BlockSpec((1, P), lambda i: (0, 0)),
            pl.BlockSpec((P, M), lambda i: (0, 0)),
            pl.BlockSpec((H, M), lambda i: (0, 0)),
            pl.BlockSpec((8, H), lambda i: (0, 0)),
            pl.BlockSpec((1, M), lambda i: (0, 0)),
        ],
        out_specs=pl.BlockSpec((B, M), lambda i: (0, 0)),
        out_shape=jax.ShapeDtypeStruct((B, M), jnp.float32),
    )


def _build_mlp2(B, Gp, M):
    TG = 1024
    grid = (Gp // TG,)

    def body(h1_ref, w2_ref, b2_ref, out_ref):
        out_ref[...] = jnp.dot(h1_ref[...], w2_ref[...],
                               preferred_element_type=jnp.float32) + b2_ref[...]

    return pl.pallas_call(
        body,
        grid=grid,
        in_specs=[
            pl.BlockSpec((B, M), lambda i: (0, 0)),
            pl.BlockSpec((M, TG), lambda i: (0, i)),
            pl.BlockSpec((1, TG), lambda i: (0, i)),
        ],
        out_specs=pl.BlockSpec((B, TG), lambda i: (0, i)),
        out_shape=jax.ShapeDtypeStruct((B, Gp), jnp.float32),
    )


def kernel(x, edge_index, ctrl, pert, pos, Wq, bq, Wk, bk, Wv, bv,
           Wskip, bskip, W1, b1, Wp, bp, Wm1, bm1, Wm2, bm2):
    N, D = x.shape
    E = edge_index.shape[1]
    H = Wq.shape[1]
    B, G = ctrl.shape
    P = pert.shape[1]
    M = Wm1.shape[1]
    HH = H // 2
    NP = ((N + NW * 16 - 1) // (NW * 16)) * (NW * 16)
    ND = NP
    SLICE = ND // NSUB
    Ep = ((E + NW * CH - 1) // (NW * CH)) * (NW * CH)
    EPW = Ep // NW
    CHW = EPW // CH
    ESUB = Ep // NSUB
    CHS = ESUB // CH

    xp = jnp.pad(x, ((0, NP - N), (0, 0)))
    src = jnp.concatenate([edge_index[0],
                           jnp.zeros((Ep - E,), jnp.int32)])
    dst = jnp.concatenate([edge_index[1],
                           jnp.full((Ep - E,), NP - 2, jnp.int32)])
    CH1 = 64
    CHW1 = EPW // CH1
    dstw = dst.reshape(NW, CHW, CH)
    srcw = src.reshape(NW, CHW, CH)
    dstw1 = dst.reshape(NW, CHW1, CH1)
    srcw1 = src.reshape(NW, CHW1, CH1)
    CH2 = 64
    CHS2 = ESUB // CH2
    dsts = dst.reshape(NSUB, CHS2, CH2)
    srcs = src.reshape(NSUB, CHS2, CH2)

    wbig = jnp.concatenate([Wq, Wk, Wv, Wskip + W1], axis=1)
    bbig = jnp.concatenate([bq, bk, bv, bskip + b1])[None, :]
    q, k, v0, v1, sx = _build_proj(NP, D, H)(xp, wbig, bbig)

    alpha, mx = _build_k1(NP, Ep, H, CH1, CHW1, EPW)(q, k, dstw1, srcw1)
    znd = jnp.zeros((ND,), jnp.float32)
    ex, den2 = _build_k2(ND, SLICE, CHW, EPW)(alpha, dstw, mx, znd)
    vcat = jnp.concatenate([v0, v1], axis=0)
    zagg = jnp.zeros((SLICE, HH), jnp.float32)
    exs = ex.reshape(NSUB, ESUB)
    aggc = _build_k3(N, NP, ND, SLICE, HH, CHS2, CH2, ESUB)(
        vcat, exs, dsts, srcs, den2, zagg)

    pooled = _build_pool(N, H)(aggc[0], aggc[1], sx)

    Gp = ((G + 1023) // 1024) * 1024
    ctrl_p = jnp.pad(ctrl, ((0, 0), (0, Gp - G)))
    w1a = jnp.pad(Wm1[:G], ((0, Gp - G), (0, 0)))
    w1c = Wm1[G:G + H]
    w1b = Wm1[G + H:]
    h1 = _build_mlp1(B, Gp, H, P, M)(ctrl_p, w1a, pert, Wp, bp[None], w1b,
                                     w1c, pooled, bm1[None])
    w2p = jnp.pad(Wm2, ((0, 0), (0, Gp - G)))
    b2p = jnp.pad(bm2, (0, Gp - G))
    out = _build_mlp2(B, Gp, M)(h1, w2p, b2p[None])
    return out[:, :G]


# speedup vs baseline: 1.0518x; 1.0518x over previous
import functools

import jax
import jax.numpy as jnp
from jax import lax
from jax.experimental import pallas as pl
from jax.experimental.pallas import tpu as pltpu
from jax.experimental.pallas import tpu_sc as plsc

NEG_BIG = -3.0e38
_SC_PARAMS = pltpu.CompilerParams(use_tc_tiling_on_sc=False,
                                  needs_layout_passes=False)
CH = 128
NW = 32
NSUB = 16


def _tree_sum(vs):
    vs = list(vs)
    while len(vs) > 1:
        nxt = [vs[i] + vs[i + 1] for i in range(0, len(vs) - 1, 2)]
        if len(vs) % 2:
            nxt.append(vs[-1])
        vs = nxt
    return vs[0]


def _build_proj(NP, D, H):
    TN = 512
    grid = (NP // TN,)

    def body(x_ref, w_ref, b_ref, qk_ref, v0_ref, v1_ref, s_ref):
        res = jnp.dot(x_ref[...], w_ref[...],
                      preferred_element_type=jnp.float32) + b_ref[...]
        qk_ref[0] = res[:, 0:H]
        qk_ref[1] = res[:, H:2 * H]
        qk_ref[2] = res[:, 0:H]
        qk_ref[3] = res[:, H:2 * H]
        v0_ref[...] = res[:, 2 * H:2 * H + H // 2]
        v1_ref[...] = res[:, 2 * H + H // 2:3 * H]
        s_ref[...] = res[:, 3 * H:4 * H]

    return pl.pallas_call(
        body,
        grid=grid,
        in_specs=[
            pl.BlockSpec((TN, D), lambda i: (i, 0)),
            pl.BlockSpec((D, 4 * H), lambda i: (0, 0)),
            pl.BlockSpec((1, 4 * H), lambda i: (0, 0)),
        ],
        out_specs=[
            pl.BlockSpec((4, TN, H), lambda i: (0, i, 0)),
            pl.BlockSpec((TN, H // 2), lambda i: (i, 0)),
            pl.BlockSpec((TN, H // 2), lambda i: (i, 0)),
            pl.BlockSpec((TN, H), lambda i: (i, 0)),
        ],
        out_shape=[
            jax.ShapeDtypeStruct((4, NP, H), jnp.float32),
            jax.ShapeDtypeStruct((NP, H // 2), jnp.float32),
            jax.ShapeDtypeStruct((NP, H // 2), jnp.float32),
            jax.ShapeDtypeStruct((NP, H), jnp.float32),
        ],
    )


def _build_k1(NP, Ep, H, CH1, CHW1, EPW):
    inv_sqrt_h = 1.0 / (H ** 0.5)
    mesh = plsc.VectorSubcoreMesh(core_axis_name="c", subcore_axis_name="s")

    @functools.partial(
        pl.kernel,
        out_type=(jax.ShapeDtypeStruct((NW, EPW), jnp.float32),
                  jax.ShapeDtypeStruct((NW, 16), jnp.float32)),
        mesh=mesh,
        compiler_params=_SC_PARAMS,
        scratch_types=[
            pltpu.VMEM((CHW1, CH1), jnp.int32),
            pltpu.VMEM((CHW1, CH1), jnp.int32),
            pltpu.VMEM((CH1, H), jnp.float32),
            pltpu.VMEM((CH1, H), jnp.float32),
            pltpu.VMEM((CH1, H), jnp.float32),
            pltpu.VMEM((CH1, H), jnp.float32),
            pltpu.VMEM((EPW,), jnp.float32),
            pltpu.VMEM((16, 16), jnp.float32),
            pltpu.VMEM((16,), jnp.float32),
            pltpu.SemaphoreType.DMA,
            pltpu.SemaphoreType.DMA,
            pltpu.SemaphoreType.DMA,
            pltpu.SemaphoreType.DMA,
        ],
    )
    def k1(qk_hbm, dst_hbm, src_hbm, alpha_hbm, mx_hbm,
           dst2d, src2d, qr0, kr0, qr1, kr1, alphabig, tbuf, mxbuf,
           sq0, sk0, sq1, sk1):
        c = lax.axis_index("c")
        s = lax.axis_index("s")
        wid = s * 2 + c
        pltpu.sync_copy(dst_hbm.at[wid], dst2d)
        pltpu.sync_copy(src_hbm.at[wid], src2d)
        iota = jnp.arange(16, dtype=jnp.int32)
        inv = jnp.float32(inv_sqrt_h)
        qoff = 2 * c * NP
        koff = qoff + NP

        def offbody(t, carry):
            for j in range(CH1 // 16):
                sl = pl.ds(j * 16, 16)
                dst2d[t, sl] = dst2d[t, sl] + qoff
                src2d[t, sl] = src2d[t, sl] + koff
            return carry

        lax.fori_loop(0, CHW1, offbody, jnp.int32(0))
        bufs = ((qr0, kr0, sq0, sk0), (qr1, kr1, sq1, sk1))

        def issue(t, b):
            qr, kr, sq, sk = bufs[b]
            pltpu.async_copy(qk_hbm.at[dst2d.at[t]], qr, sq)
            pltpu.async_copy(qk_hbm.at[src2d.at[t]], kr, sk)

        def wait(t, b):
            qr, kr, sq, sk = bufs[b]
            pltpu.make_async_copy(qk_hbm.at[dst2d.at[t]], qr, sq).wait()
            pltpu.make_async_copy(qk_hbm.at[src2d.at[t]], kr, sk).wait()

        def compute(t, b, mxv):
            qr, kr, _, _ = bufs[b]
            for grp in range(CH1 // 16):

                def ebody(e, carry):
                    r = grp * 16 + e
                    ps = [qr[r, pl.ds(j * 16, 16)] * kr[r, pl.ds(j * 16, 16)]
                          for j in range(H // 16)]
                    tbuf[e, pl.ds(0, 16)] = _tree_sum(ps)
                    return carry

                lax.fori_loop(0, 16, ebody, jnp.int32(0))
                cols = [plsc.load_gather(tbuf, [iota, jnp.full((16,), j, jnp.int32)])
                        for j in range(16)]
                a16 = _tree_sum(cols) * inv
                alphabig[pl.ds(t * CH1 + grp * 16, 16)] = a16
                mxv = jnp.maximum(mxv, a16)
            return mxv

        issue(0, 0)

        def pair_body(p, mxv):
            t0 = p * 2
            t1 = t0 + 1
            issue(t1, 1)
            wait(t0, 0)
            mxv = compute(t0, 0, mxv)

            @pl.when(t0 + 2 < CHW1)
            def _():
                issue(t0 + 2, 0)

            wait(t1, 1)
            mxv = compute(t1, 1, mxv)
            return mxv

        mxv = lax.fori_loop(0, CHW1 // 2, pair_body,
                            jnp.full((16,), NEG_BIG, jnp.float32))
        mxbuf[...] = mxv
        pltpu.sync_copy(alphabig, alpha_hbm.at[wid])
        pltpu.sync_copy(mxbuf, mx_hbm.at[wid])

    return k1


def _build_k2(ND, SLICE, CHW, EPW):
    mesh = plsc.VectorSubcoreMesh(core_axis_name="c", subcore_axis_name="s")

    @functools.partial(
        pl.kernel,
        out_type=(jax.ShapeDtypeStruct((NW, EPW), jnp.float32),
                  jax.ShapeDtypeStruct((2, ND), jnp.float32)),
        mesh=mesh,
        compiler_params=_SC_PARAMS,
        scratch_types=[
            pltpu.VMEM((NW, 16), jnp.float32),
            pltpu.VMEM((CHW, CH), jnp.int32),
            pltpu.VMEM((EPW,), jnp.float32),
            pltpu.VMEM((EPW,), jnp.float32),
            pltpu.VMEM_SHARED((ND,), jnp.float32),
        ],
    )
    def k2(alpha_hbm, dst_hbm, mx_hbm, znd_hbm, ex_hbm, den_hbm,
           mxbuf, dst2d, alphabig, exbig, denom_sp):
        c = lax.axis_index("c")
        s = lax.axis_index("s")
        wid = s * 2 + c
        pltpu.sync_copy(mx_hbm, mxbuf)

        def mbody(i, m):
            return jnp.maximum(m, mxbuf[i])

        m = lax.fori_loop(0, NW, mbody, jnp.full((16,), NEG_BIG, jnp.float32))
        cmax = jnp.max(m)
        cvec = jnp.full((16,), cmax)
        pltpu.sync_copy(znd_hbm.at[pl.ds(s * SLICE, SLICE)],
                        denom_sp.at[pl.ds(s * SLICE, SLICE)])
        pltpu.sync_copy(alpha_hbm.at[wid], alphabig)
        pltpu.sync_copy(dst_hbm.at[wid], dst2d)

        def gbody(g, carry):
            sl = pl.ds(g * 16, 16)
            exbig[sl] = jnp.exp(alphabig[sl] - cvec)
            return carry

        lax.fori_loop(0, EPW // 16, gbody, jnp.int32(0))
        pltpu.sync_copy(exbig, ex_hbm.at[wid])
        plsc.subcore_barrier()

        def sbody(t, carry):
            pltpu.sync_copy(exbig.at[pl.ds(t * CH, CH)],
                            denom_sp.at[dst2d.at[t]], add=True)
            return carry

        lax.fori_loop(0, CHW, sbody, jnp.int32(0))
        plsc.subcore_barrier()
        pltpu.sync_copy(denom_sp.at[pl.ds(s * SLICE, SLICE)],
                        den_hbm.at[c, pl.ds(s * SLICE, SLICE)])

    return k2


def _build_k3(N, NP, ND, SLICE, HH, CHS, ESUB):
    last_rows = N - (NSUB - 1) * SLICE
    GC = 8
    NG = CHS // GC
    DB = ND // 4
    mesh = plsc.VectorSubcoreMesh(core_axis_name="c", subcore_axis_name="s")

    @functools.partial(
        pl.kernel,
        out_type=jax.ShapeDtypeStruct((2, N, HH), jnp.float32),
        mesh=mesh,
        compiler_params=_SC_PARAMS,
        scratch_types=[
            pltpu.VMEM((ND,), jnp.float32),
            pltpu.VMEM((DB,), jnp.float32),
            pltpu.VMEM((GC, CH), jnp.int32),
            pltpu.VMEM((GC, CH), jnp.int32),
            pltpu.VMEM((GC * CH,), jnp.float32),
            pltpu.VMEM((CH,), jnp.float32),
            pltpu.VMEM((CH, HH), jnp.float32),
            pltpu.VMEM((CH, HH), jnp.float32),
            pltpu.VMEM_SHARED((ND, HH), jnp.float32),
            pltpu.SemaphoreType.DMA,
            pltpu.SemaphoreType.DMA,
        ],
    )
    def k3(vcat_hbm, ex_hbm, dst_hbm, src_hbm, den_hbm, zagg_hbm, agg_hbm,
           rdenom, dbuf, dstg, srcg, exg, wbuf, vr0, vr1, agg_sp, sg0, sg1):
        c = lax.axis_index("c")
        s = lax.axis_index("s")
        srcoff = c * NP
        pltpu.sync_copy(den_hbm.at[0], rdenom)
        for blk in range(4):
            pltpu.sync_copy(den_hbm.at[1, pl.ds(blk * DB, DB)], dbuf)

            def rbody(i, carry, _blk=blk):
                sl16 = pl.ds(_blk * DB + i * 16, 16)
                rdenom[sl16] = 1.0 / (rdenom[sl16] + dbuf[pl.ds(i * 16, 16)]
                                      + jnp.float32(1e-16))
                return carry

            lax.fori_loop(0, DB // 16, rbody, jnp.int32(0))
        pltpu.sync_copy(zagg_hbm, agg_sp.at[pl.ds(s * SLICE, SLICE)])
        plsc.subcore_barrier()
        bufs = ((vr0, sg0), (vr1, sg1))

        def issue(t, b):
            vr, sg = bufs[b]
            pltpu.async_copy(vcat_hbm.at[srcg.at[t]], vr, sg)

        def wait(t, b):
            vr, sg = bufs[b]
            pltpu.make_async_copy(vcat_hbm.at[srcg.at[t]], vr, sg).wait()

        def scale_scatter(t, b):
            vr, _ = bufs[b]
            for grp in range(CH // 16):
                sl = pl.ds(grp * 16, 16)
                d16 = dstg[t, sl]
                rd = plsc.load_gather(rdenom, [d16])
                wbuf[sl] = exg[pl.ds(t * CH + grp * 16, 16)] * rd

            def ebody(e, carry3):
                wsp = plsc.load_gather(wbuf, [jnp.full((16,), e, jnp.int32)])
                for cb in range(HH // 16):
                    slc = pl.ds(cb * 16, 16)
                    vr[e, slc] = vr[e, slc] * wsp
                return carry3

            lax.fori_loop(0, CH, ebody, jnp.int32(0), unroll=2)
            pltpu.sync_copy(vr, agg_sp.at[dstg.at[t]], add=True)

        def group_body(gi, carry):
            pltpu.sync_copy(dst_hbm.at[s, pl.ds(gi * GC, GC)], dstg)
            pltpu.sync_copy(src_hbm.at[s, pl.ds(gi * GC, GC)], srcg)
            pltpu.sync_copy(ex_hbm.at[s, pl.ds(gi * GC * CH, GC * CH)], exg)

            def offbody(t, carry2):
                for j in range(CH // 16):
                    sl = pl.ds(j * 16, 16)
                    srcg[t, sl] = srcg[t, sl] + srcoff
                return carry2

            lax.fori_loop(0, GC, offbody, jnp.int32(0))
            issue(0, 0)

            def pair_body(p, carry2):
                t0 = p * 2
                t1 = t0 + 1
                issue(t1, 1)
                wait(t0, 0)
                scale_scatter(t0, 0)

                @pl.when(t0 + 2 < GC)
                def _():
                    issue(t0 + 2, 0)

                wait(t1, 1)
                scale_scatter(t1, 1)
                return carry2

            lax.fori_loop(0, GC // 2, pair_body, jnp.int32(0))
            return carry

        lax.fori_loop(0, NG, group_body, jnp.int32(0))
        plsc.subcore_barrier()

        @pl.when(s != NSUB - 1)
        def _():
            pltpu.sync_copy(agg_sp.at[pl.ds(s * SLICE, SLICE)],
                            agg_hbm.at[c, pl.ds(s * SLICE, SLICE)])

        @pl.when(s == NSUB - 1)
        def _():
            pltpu.sync_copy(
                agg_sp.at[pl.ds((NSUB - 1) * SLICE, last_rows)],
                agg_hbm.at[c, pl.ds((NSUB - 1) * SLICE, last_rows)])

    return k3


def _build_pool(N, H):
    TN = 400
    grid = (N // TN,)

    def body(a0_ref, a1_ref, sx_ref, out_ref):
        i = pl.program_id(0)

        @pl.when(i == 0)
        def _():
            out_ref[...] = jnp.full_like(out_ref, NEG_BIG)

        h = jnp.concatenate([a0_ref[...], a1_ref[...]], axis=1) + sx_ref[...]
        m = jnp.max(h, axis=0, keepdims=True)
        out_ref[...] = jnp.maximum(out_ref[...], jnp.broadcast_to(m, out_ref.shape))

    return pl.pallas_call(
        body,
        grid=grid,
        in_specs=[
            pl.BlockSpec((TN, H // 2), lambda i: (i, 0)),
            pl.BlockSpec((TN, H // 2), lambda i: (i, 0)),
            pl.BlockSpec((TN, H), lambda i: (i, 0)),
        ],
        out_specs=pl.BlockSpec((8, H), lambda i: (0, 0)),
        out_shape=jax.ShapeDtypeStruct((8, H), jnp.float32),
    )


def _build_mlp1(B, Gp, H, P, M):
    TK = 1024
    nk = Gp // TK
    grid = (nk,)

    def body(ctrl_ref, w1a_ref, pert_ref, wp_ref, bp_ref, w1b_ref, w1c_ref,
             pooled_ref, bm1_ref, out_ref):
        i = pl.program_id(0)

        @pl.when(i == 0)
        def _():
            out_ref[...] = jnp.zeros_like(out_ref)

        out_ref[...] += jnp.dot(ctrl_ref[...], w1a_ref[...],
                                preferred_element_type=jnp.float32)

        @pl.when(i == nk - 1)
        def _():
            emb = jnp.dot(pert_ref[...], wp_ref[...],
                          preferred_element_type=jnp.float32) + bp_ref[...]
            acc2 = jnp.dot(emb, w1b_ref[...], preferred_element_type=jnp.float32)
            t = jnp.dot(pooled_ref[0:1, :], w1c_ref[...],
                        preferred_element_type=jnp.float32)
            z = out_ref[...] + acc2 + t + bm1_ref[...]
            out_ref[...] = jax.nn.softplus(z)

    return pl.pallas_call(
        body,
        grid=grid,
        in_specs=[
            pl.BlockSpec((B, TK), lambda i: (0, i)),
            pl.BlockSpec((TK, M), lambda i: (i, 0)),
            pl.BlockSpec((B, P), lambda i: (0, 0)),
            pl.BlockSpec((P, P), lambda i: (0, 0)),
            pl.BlockSpec((1, P), lambda i: (0, 0)),
            pl.BlockSpec((P, M), lambda i: (0, 0)),
            pl.BlockSpec((H, M), lambda i: (0, 0)),
            pl.BlockSpec((8, H), lambda i: (0, 0)),
            pl.BlockSpec((1, M), lambda i: (0, 0)),
        ],
        out_specs=pl.BlockSpec((B, M), lambda i: (0, 0)),
        out_shape=jax.ShapeDtypeStruct((B, M), jnp.float32),
    )


def _build_mlp2(B, Gp, M):
    TG = 1024
    grid = (Gp // TG,)

    def body(h1_ref, w2_ref, b2_ref, out_ref):
        out_ref[...] = jnp.dot(h1_ref[...], w2_ref[...],
                               preferred_element_type=jnp.float32) + b2_ref[...]

    return pl.pallas_call(
        body,
        grid=grid,
        in_specs=[
            pl.BlockSpec((B, M), lambda i: (0, 0)),
            pl.BlockSpec((M, TG), lambda i: (0, i)),
            pl.BlockSpec((1, TG), lambda i: (0, i)),
        ],
        out_specs=pl.BlockSpec((B, TG), lambda i: (0, i)),
        out_shape=jax.ShapeDtypeStruct((B, Gp), jnp.float32),
    )


def kernel(x, edge_index, ctrl, pert, pos, Wq, bq, Wk, bk, Wv, bv,
           Wskip, bskip, W1, b1, Wp, bp, Wm1, bm1, Wm2, bm2):
    N, D = x.shape
    E = edge_index.shape[1]
    H = Wq.shape[1]
    B, G = ctrl.shape
    P = pert.shape[1]
    M = Wm1.shape[1]
    HH = H // 2
    NP = ((N + NW * 16 - 1) // (NW * 16)) * (NW * 16)
    ND = NP
    SLICE = ND // NSUB
    Ep = ((E + NW * CH - 1) // (NW * CH)) * (NW * CH)
    EPW = Ep // NW
    CHW = EPW // CH
    ESUB = Ep // NSUB
    CHS = ESUB // CH

    xp = jnp.pad(x, ((0, NP - N), (0, 0)))
    src = jnp.concatenate([edge_index[0],
                           jnp.zeros((Ep - E,), jnp.int32)])
    dst = jnp.concatenate([edge_index[1],
                           jnp.full((Ep - E,), NP - 2, jnp.int32)])
    CH1 = 64
    CHW1 = EPW // CH1
    dstw = dst.reshape(NW, CHW, CH)
    srcw = src.reshape(NW, CHW, CH)
    dstw1 = dst.reshape(NW, CHW1, CH1)
    srcw1 = src.reshape(NW, CHW1, CH1)
    dsts = dst.reshape(NSUB, CHS, CH)
    srcs = src.reshape(NSUB, CHS, CH)

    wbig = jnp.concatenate([Wq, Wk, Wv, Wskip + W1], axis=1)
    bbig = jnp.concatenate([bq, bk, bv, bskip + b1])[None, :]
    qk4, v0, v1, sx = _build_proj(NP, D, H)(xp, wbig, bbig)
    qkflat = qk4.reshape(4 * NP, H)

    alpha, mx = _build_k1(NP, Ep, H, CH1, CHW1, EPW)(qkflat, dstw1, srcw1)
    znd = jnp.zeros((ND,), jnp.float32)
    ex, den2 = _build_k2(ND, SLICE, CHW, EPW)(alpha, dstw, mx, znd)
    vcat = jnp.concatenate([v0, v1], axis=0)
    zagg = jnp.zeros((SLICE, HH), jnp.float32)
    exs = ex.reshape(NSUB, ESUB)
    aggc = _build_k3(N, NP, ND, SLICE, HH, CHS, ESUB)(
        vcat, exs, dsts, srcs, den2, zagg)

    pooled = _build_pool(N, H)(aggc[0], aggc[1], sx)

    Gp = ((G + 1023) // 1024) * 1024
    ctrl_p = jnp.pad(ctrl, ((0, 0), (0, Gp - G)))
    w1a = jnp.pad(Wm1[:G], ((0, Gp - G), (0, 0)))
    w1c = Wm1[G:G + H]
    w1b = Wm1[G + H:]
    h1 = _build_mlp1(B, Gp, H, P, M)(ctrl_p, w1a, pert, Wp, bp[None], w1b,
                                     w1c, pooled, bm1[None])
    w2p = jnp.pad(Wm2, ((0, 0), (0, Gp - G)))
    b2p = jnp.pad(bm2, (0, Gp - G))
    out = _build_mlp2(B, Gp, M)(h1, w2p, b2p[None])
    return out[:, :G]


# speedup vs baseline: 1.6407x; 1.5599x over previous
import functools

import jax
import jax.numpy as jnp
from jax import lax
from jax.experimental import pallas as pl
from jax.experimental.pallas import tpu as pltpu
from jax.experimental.pallas import tpu_sc as plsc

NEG_BIG = -3.0e38
_SC_PARAMS = pltpu.CompilerParams(use_tc_tiling_on_sc=False,
                                  needs_layout_passes=False)
CH = 128
NW = 32
NSUB = 16


def _tree_sum(vs):
    vs = list(vs)
    while len(vs) > 1:
        nxt = [vs[i] + vs[i + 1] for i in range(0, len(vs) - 1, 2)]
        if len(vs) % 2:
            nxt.append(vs[-1])
        vs = nxt
    return vs[0]


def _build_proj(NP, D, H):
    TN = 512
    grid = (NP // TN,)

    def body(x_ref, w_ref, b_ref, qk_ref, v0_ref, v1_ref, s_ref):
        res = jnp.dot(x_ref[...], w_ref[...],
                      preferred_element_type=jnp.float32) + b_ref[...]
        qk_ref[0] = res[:, 0:H]
        qk_ref[1] = res[:, H:2 * H]
        qk_ref[2] = res[:, 0:H]
        qk_ref[3] = res[:, H:2 * H]
        v0_ref[...] = res[:, 2 * H:2 * H + H // 2]
        v1_ref[...] = res[:, 2 * H + H // 2:3 * H]
        s_ref[...] = res[:, 3 * H:4 * H]

    return pl.pallas_call(
        body,
        grid=grid,
        in_specs=[
            pl.BlockSpec((TN, D), lambda i: (i, 0)),
            pl.BlockSpec((D, 4 * H), lambda i: (0, 0)),
            pl.BlockSpec((1, 4 * H), lambda i: (0, 0)),
        ],
        out_specs=[
            pl.BlockSpec((4, TN, H), lambda i: (0, i, 0)),
            pl.BlockSpec((TN, H // 2), lambda i: (i, 0)),
            pl.BlockSpec((TN, H // 2), lambda i: (i, 0)),
            pl.BlockSpec((TN, H), lambda i: (i, 0)),
        ],
        out_shape=[
            jax.ShapeDtypeStruct((4, NP, H), jnp.float32),
            jax.ShapeDtypeStruct((NP, H // 2), jnp.float32),
            jax.ShapeDtypeStruct((NP, H // 2), jnp.float32),
            jax.ShapeDtypeStruct((NP, H), jnp.float32),
        ],
    )


def _build_k1(NP, Ep, H, CH1, CHW1, EPW):
    inv_sqrt_h = 1.0 / (H ** 0.5)
    mesh = plsc.VectorSubcoreMesh(core_axis_name="c", subcore_axis_name="s")

    @functools.partial(
        pl.kernel,
        out_type=(jax.ShapeDtypeStruct((NW, EPW), jnp.float32),
                  jax.ShapeDtypeStruct((NW, 16), jnp.float32)),
        mesh=mesh,
        compiler_params=_SC_PARAMS,
        scratch_types=[
            pltpu.VMEM((CHW1, CH1), jnp.int32),
            pltpu.VMEM((CHW1, CH1), jnp.int32),
            pltpu.VMEM((CH1, H), jnp.float32),
            pltpu.VMEM((CH1, H), jnp.float32),
            pltpu.VMEM((CH1, H), jnp.float32),
            pltpu.VMEM((CH1, H), jnp.float32),
            pltpu.VMEM((EPW,), jnp.float32),
            pltpu.VMEM((16, 16), jnp.float32),
            pltpu.VMEM((16,), jnp.float32),
            pltpu.SemaphoreType.DMA,
            pltpu.SemaphoreType.DMA,
            pltpu.SemaphoreType.DMA,
            pltpu.SemaphoreType.DMA,
        ],
    )
    def k1(qk_hbm, dst_hbm, src_hbm, alpha_hbm, mx_hbm,
           dst2d, src2d, qr0, kr0, qr1, kr1, alphabig, tbuf, mxbuf,
           sq0, sk0, sq1, sk1):
        c = lax.axis_index("c")
        s = lax.axis_index("s")
        wid = s * 2 + c
        pltpu.sync_copy(dst_hbm.at[wid], dst2d)
        pltpu.sync_copy(src_hbm.at[wid], src2d)
        iota = jnp.arange(16, dtype=jnp.int32)
        inv = jnp.float32(inv_sqrt_h)
        qoff = 2 * c * NP
        koff = qoff + NP

        def offbody(t, carry):
            for j in range(CH1 // 16):
                sl = pl.ds(j * 16, 16)
                dst2d[t, sl] = dst2d[t, sl] + qoff
                src2d[t, sl] = src2d[t, sl] + koff
            return carry

        lax.fori_loop(0, CHW1, offbody, jnp.int32(0))
        bufs = ((qr0, kr0, sq0, sk0), (qr1, kr1, sq1, sk1))

        def issue(t, b):
            qr, kr, sq, sk = bufs[b]
            pltpu.async_copy(qk_hbm.at[dst2d.at[t]], qr, sq)
            pltpu.async_copy(qk_hbm.at[src2d.at[t]], kr, sk)

        def wait(t, b):
            qr, kr, sq, sk = bufs[b]
            pltpu.make_async_copy(qk_hbm.at[dst2d.at[t]], qr, sq).wait()
            pltpu.make_async_copy(qk_hbm.at[src2d.at[t]], kr, sk).wait()

        def compute(t, b, mxv):
            qr, kr, _, _ = bufs[b]
            for grp in range(CH1 // 16):

                def ebody(e, carry):
                    r = grp * 16 + e
                    ps = [qr[r, pl.ds(j * 16, 16)] * kr[r, pl.ds(j * 16, 16)]
                          for j in range(H // 16)]
                    tbuf[e, pl.ds(0, 16)] = _tree_sum(ps)
                    return carry

                lax.fori_loop(0, 16, ebody, jnp.int32(0))
                cols = [plsc.load_gather(tbuf, [iota, jnp.full((16,), j, jnp.int32)])
                        for j in range(16)]
                a16 = _tree_sum(cols) * inv
                alphabig[pl.ds(t * CH1 + grp * 16, 16)] = a16
                mxv = jnp.maximum(mxv, a16)
            return mxv

        issue(0, 0)

        def pair_body(p, mxv):
            t0 = p * 2
            t1 = t0 + 1
            issue(t1, 1)
            wait(t0, 0)
            mxv = compute(t0, 0, mxv)

            @pl.when(t0 + 2 < CHW1)
            def _():
                issue(t0 + 2, 0)

            wait(t1, 1)
            mxv = compute(t1, 1, mxv)
            return mxv

        mxv = lax.fori_loop(0, CHW1 // 2, pair_body,
                            jnp.full((16,), NEG_BIG, jnp.float32))
        mxbuf[...] = mxv
        pltpu.sync_copy(alphabig, alpha_hbm.at[wid])
        pltpu.sync_copy(mxbuf, mx_hbm.at[wid])

    return k1


def _build_k2(ND, SLICE, CHW, EPW):
    mesh = plsc.VectorSubcoreMesh(core_axis_name="c", subcore_axis_name="s")

    @functools.partial(
        pl.kernel,
        out_type=(jax.ShapeDtypeStruct((NW, EPW), jnp.float32),
                  jax.ShapeDtypeStruct((2, ND), jnp.float32)),
        mesh=mesh,
        compiler_params=_SC_PARAMS,
        scratch_types=[
            pltpu.VMEM((NW, 16), jnp.float32),
            pltpu.VMEM((CHW, CH), jnp.int32),
            pltpu.VMEM((EPW,), jnp.float32),
            pltpu.VMEM((EPW,), jnp.float32),
            pltpu.VMEM_SHARED((ND,), jnp.float32),
        ],
    )
    def k2(alpha_hbm, dst_hbm, mx_hbm, znd_hbm, ex_hbm, den_hbm,
           mxbuf, dst2d, alphabig, exbig, denom_sp):
        c = lax.axis_index("c")
        s = lax.axis_index("s")
        wid = s * 2 + c
        pltpu.sync_copy(mx_hbm, mxbuf)

        def mbody(i, m):
            return jnp.maximum(m, mxbuf[i])

        m = lax.fori_loop(0, NW, mbody, jnp.full((16,), NEG_BIG, jnp.float32))
        cmax = jnp.max(m)
        cvec = jnp.full((16,), cmax)
        pltpu.sync_copy(znd_hbm.at[pl.ds(s * SLICE, SLICE)],
                        denom_sp.at[pl.ds(s * SLICE, SLICE)])
        pltpu.sync_copy(alpha_hbm.at[wid], alphabig)
        pltpu.sync_copy(dst_hbm.at[wid], dst2d)

        def gbody(g, carry):
            sl = pl.ds(g * 16, 16)
            exbig[sl] = jnp.exp(alphabig[sl] - cvec)
            return carry

        lax.fori_loop(0, EPW // 16, gbody, jnp.int32(0))
        pltpu.sync_copy(exbig, ex_hbm.at[wid])
        plsc.subcore_barrier()

        def sbody(t, carry):
            pltpu.sync_copy(exbig.at[pl.ds(t * CH, CH)],
                            denom_sp.at[dst2d.at[t]], add=True)
            return carry

        lax.fori_loop(0, CHW, sbody, jnp.int32(0))
        plsc.subcore_barrier()
        pltpu.sync_copy(denom_sp.at[pl.ds(s * SLICE, SLICE)],
                        den_hbm.at[c, pl.ds(s * SLICE, SLICE)])

    return k2


def _build_k3(N, NP, ND, SLICE, HH, CHS, ESUB):
    last_rows = N - (NSUB - 1) * SLICE
    GC = 8
    NG = CHS // GC
    DB = ND // 4
    mesh = plsc.VectorSubcoreMesh(core_axis_name="c", subcore_axis_name="s")

    @functools.partial(
        pl.kernel,
        out_type=jax.ShapeDtypeStruct((2, N, HH), jnp.float32),
        mesh=mesh,
        compiler_params=_SC_PARAMS,
        scratch_types=[
            pltpu.VMEM((ND,), jnp.float32),
            pltpu.VMEM((DB,), jnp.float32),
            pltpu.VMEM((GC, CH), jnp.int32),
            pltpu.VMEM((GC, CH), jnp.int32),
            pltpu.VMEM((GC * CH,), jnp.float32),
            pltpu.VMEM((CH,), jnp.float32),
            pltpu.VMEM((CH, HH), jnp.float32),
            pltpu.VMEM((CH, HH), jnp.float32),
            pltpu.VMEM_SHARED((ND, HH), jnp.float32),
            pltpu.SemaphoreType.DMA,
            pltpu.SemaphoreType.DMA,
        ],
    )
    def k3(vcat_hbm, ex_hbm, dst_hbm, src_hbm, den_hbm, zagg_hbm, agg_hbm,
           rdenom, dbuf, dstg, srcg, exg, wbuf, vr0, vr1, agg_sp, sg0, sg1):
        c = lax.axis_index("c")
        s = lax.axis_index("s")
        srcoff = c * NP
        pltpu.sync_copy(den_hbm.at[0], rdenom)
        for blk in range(4):
            pltpu.sync_copy(den_hbm.at[1, pl.ds(blk * DB, DB)], dbuf)

            def rbody(i, carry, _blk=blk):
                sl16 = pl.ds(_blk * DB + i * 16, 16)
                rdenom[sl16] = 1.0 / (rdenom[sl16] + dbuf[pl.ds(i * 16, 16)]
                                      + jnp.float32(1e-16))
                return carry

            lax.fori_loop(0, DB // 16, rbody, jnp.int32(0))
        pltpu.sync_copy(zagg_hbm, agg_sp.at[pl.ds(s * SLICE, SLICE)])
        plsc.subcore_barrier()
        bufs = ((vr0, sg0), (vr1, sg1))

        def issue(t, b):
            vr, sg = bufs[b]
            pltpu.async_copy(vcat_hbm.at[srcg.at[t]], vr, sg)

        def wait(t, b):
            vr, sg = bufs[b]
            pltpu.make_async_copy(vcat_hbm.at[srcg.at[t]], vr, sg).wait()

        def scale_scatter(t, b):
            vr, _ = bufs[b]
            for grp in range(CH // 16):
                sl = pl.ds(grp * 16, 16)
                d16 = dstg[t, sl]
                rd = plsc.load_gather(rdenom, [d16])
                wbuf[sl] = exg[pl.ds(t * CH + grp * 16, 16)] * rd

            def ebody(e, carry3):
                wsp = plsc.load_gather(wbuf, [jnp.full((16,), e, jnp.int32)])
                for cb in range(HH // 16):
                    slc = pl.ds(cb * 16, 16)
                    vr[e, slc] = vr[e, slc] * wsp
                return carry3

            lax.fori_loop(0, CH, ebody, jnp.int32(0), unroll=2)
            pltpu.sync_copy(vr, agg_sp.at[dstg.at[t]], add=True)

        def group_body(gi, carry):
            pltpu.sync_copy(dst_hbm.at[s, pl.ds(gi * GC, GC)], dstg)
            pltpu.sync_copy(src_hbm.at[s, pl.ds(gi * GC, GC)], srcg)
            pltpu.sync_copy(ex_hbm.at[s, pl.ds(gi * GC * CH, GC * CH)], exg)

            def offbody(t, carry2):
                for j in range(CH // 16):
                    sl = pl.ds(j * 16, 16)
                    srcg[t, sl] = srcg[t, sl] + srcoff
                return carry2

            lax.fori_loop(0, GC, offbody, jnp.int32(0))
            issue(0, 0)

            def pair_body(p, carry2):
                t0 = p * 2
                t1 = t0 + 1
                issue(t1, 1)
                wait(t0, 0)
                scale_scatter(t0, 0)

                @pl.when(t0 + 2 < GC)
                def _():
                    issue(t0 + 2, 0)

                wait(t1, 1)
                scale_scatter(t1, 1)
                return carry2

            lax.fori_loop(0, GC // 2, pair_body, jnp.int32(0))
            return carry

        lax.fori_loop(0, NG, group_body, jnp.int32(0))
        plsc.subcore_barrier()

        @pl.when(s != NSUB - 1)
        def _():
            pltpu.sync_copy(agg_sp.at[pl.ds(s * SLICE, SLICE)],
                            agg_hbm.at[c, pl.ds(s * SLICE, SLICE)])

        @pl.when(s == NSUB - 1)
        def _():
            pltpu.sync_copy(
                agg_sp.at[pl.ds((NSUB - 1) * SLICE, last_rows)],
                agg_hbm.at[c, pl.ds((NSUB - 1) * SLICE, last_rows)])

    return k3


def _build_pool(N, H):
    TN = 400
    grid = (N // TN,)

    def body(a0_ref, a1_ref, sx_ref, out_ref):
        i = pl.program_id(0)

        @pl.when(i == 0)
        def _():
            out_ref[...] = jnp.full_like(out_ref, NEG_BIG)

        h = jnp.concatenate([a0_ref[...], a1_ref[...]], axis=1) + sx_ref[...]
        m = jnp.max(h, axis=0, keepdims=True)
        out_ref[...] = jnp.maximum(out_ref[...], jnp.broadcast_to(m, out_ref.shape))

    return pl.pallas_call(
        body,
        grid=grid,
        in_specs=[
            pl.BlockSpec((TN, H // 2), lambda i: (i, 0)),
            pl.BlockSpec((TN, H // 2), lambda i: (i, 0)),
            pl.BlockSpec((TN, H), lambda i: (i, 0)),
        ],
        out_specs=pl.BlockSpec((8, H), lambda i: (0, 0)),
        out_shape=jax.ShapeDtypeStruct((8, H), jnp.float32),
    )


def _build_mlp1(B, Gp, H, P, M):
    TK = 1024
    nk = Gp // TK
    grid = (nk,)

    def body(ctrl_ref, w1a_ref, pert_ref, wp_ref, bp_ref, w1b_ref, w1c_ref,
             pooled_ref, bm1_ref, out_ref):
        i = pl.program_id(0)

        @pl.when(i == 0)
        def _():
            out_ref[...] = jnp.zeros_like(out_ref)

        out_ref[...] += jnp.dot(ctrl_ref[...], w1a_ref[...],
                                preferred_element_type=jnp.float32)

        @pl.when(i == nk - 1)
        def _():
            emb = jnp.dot(pert_ref[...], wp_ref[...],
                          preferred_element_type=jnp.float32) + bp_ref[...]
            acc2 = jnp.dot(emb, w1b_ref[...], preferred_element_type=jnp.float32)
            t = jnp.dot(pooled_ref[0:1, :], w1c_ref[...],
                        preferred_element_type=jnp.float32)
            z = out_ref[...] + acc2 + t + bm1_ref[...]
            out_ref[...] = jax.nn.softplus(z)

    return pl.pallas_call(
        body,
        grid=grid,
        in_specs=[
            pl.BlockSpec((B, TK), lambda i: (0, i)),
            pl.BlockSpec((TK, M), lambda i: (i, 0)),
            pl.BlockSpec((B, P), lambda i: (0, 0)),
            pl.BlockSpec((P, P), lambda i: (0, 0)),
            pl.BlockSpec((1, P), lambda i: (0, 0)),
            pl.BlockSpec((P, M), lambda i: (0, 0)),
            pl.BlockSpec((H, M), lambda i: (0, 0)),
            pl.BlockSpec((8, H), lambda i: (0, 0)),
            pl.BlockSpec((1, M), lambda i: (0, 0)),
        ],
        out_specs=pl.BlockSpec((B, M), lambda i: (0, 0)),
        out_shape=jax.ShapeDtypeStruct((B, M), jnp.float32),
    )


def _build_mlp2(B, Gp, M):
    TG = 1024
    grid = (Gp // TG,)

    def body(h1_ref, w2_ref, b2_ref, out_ref):
        out_ref[...] = jnp.dot(h1_ref[...], w2_ref[...],
                               preferred_element_type=jnp.float32) + b2_ref[...]

    return pl.pallas_call(
        body,
        grid=grid,
        in_specs=[
            pl.BlockSpec((B, M), lambda i: (0, 0)),
            pl.BlockSpec((M, TG), lambda i: (0, i)),
            pl.BlockSpec((1, TG), lambda i: (0, i)),
        ],
        out_specs=pl.BlockSpec((B, TG), lambda i: (0, i)),
        out_shape=jax.ShapeDtypeStruct((B, Gp), jnp.float32),
    )


def kernel(x, edge_index, ctrl, pert, pos, Wq, bq, Wk, bk, Wv, bv,
           Wskip, bskip, W1, b1, Wp, bp, Wm1, bm1, Wm2, bm2):
    N, D = x.shape
    E = edge_index.shape[1]
    H = Wq.shape[1]
    B, G = ctrl.shape
    P = pert.shape[1]
    M = Wm1.shape[1]
    HH = H // 2
    NP = ((N + NW * 16 - 1) // (NW * 16)) * (NW * 16)
    ND = NP
    SLICE = ND // NSUB
    Ep = ((E + NW * CH - 1) // (NW * CH)) * (NW * CH)
    EPW = Ep // NW
    CHW = EPW // CH
    ESUB = Ep // NSUB
    CHS = ESUB // CH

    xp = jnp.pad(x, ((0, NP - N), (0, 0)))
    pad_ids = N + (jnp.arange(Ep - E, dtype=jnp.int32) % (NP - N))
    src = jnp.concatenate([edge_index[0], pad_ids])
    dst = jnp.concatenate([edge_index[1], pad_ids])
    CH1 = 64
    CHW1 = EPW // CH1
    dstw = dst.reshape(NW, CHW, CH)
    srcw = src.reshape(NW, CHW, CH)
    dstw1 = dst.reshape(NW, CHW1, CH1)
    srcw1 = src.reshape(NW, CHW1, CH1)
    dsts = dst.reshape(NSUB, CHS, CH)
    srcs = src.reshape(NSUB, CHS, CH)

    wbig = jnp.concatenate([Wq, Wk, Wv, Wskip + W1], axis=1)
    bbig = jnp.concatenate([bq, bk, bv, bskip + b1])[None, :]
    qk4, v0, v1, sx = _build_proj(NP, D, H)(xp, wbig, bbig)
    qkflat = qk4.reshape(4 * NP, H)

    alpha, mx = _build_k1(NP, Ep, H, CH1, CHW1, EPW)(qkflat, dstw1, srcw1)
    znd = jnp.zeros((ND,), jnp.float32)
    ex, den2 = _build_k2(ND, SLICE, CHW, EPW)(alpha, dstw, mx, znd)
    vcat = jnp.concatenate([v0, v1], axis=0)
    zagg = jnp.zeros((SLICE, HH), jnp.float32)
    exs = ex.reshape(NSUB, ESUB)
    aggc = _build_k3(N, NP, ND, SLICE, HH, CHS, ESUB)(
        vcat, exs, dsts, srcs, den2, zagg)

    pooled = _build_pool(N, H)(aggc[0], aggc[1], sx)

    Gp = ((G + 1023) // 1024) * 1024
    ctrl_p = jnp.pad(ctrl, ((0, 0), (0, Gp - G)))
    w1a = jnp.pad(Wm1[:G], ((0, Gp - G), (0, 0)))
    w1c = Wm1[G:G + H]
    w1b = Wm1[G + H:]
    h1 = _build_mlp1(B, Gp, H, P, M)(ctrl_p, w1a, pert, Wp, bp[None], w1b,
                                     w1c, pooled, bm1[None])
    w2p = jnp.pad(Wm2, ((0, 0), (0, Gp - G)))
    b2p = jnp.pad(bm2, (0, Gp - G))
    out = _build_mlp2(B, Gp, M)(h1, w2p, b2p[None])
    return out[:, :G]


# speedup vs baseline: 1.7721x; 1.0801x over previous
import functools

import jax
import jax.numpy as jnp
from jax import lax
from jax.experimental import pallas as pl
from jax.experimental.pallas import tpu as pltpu
from jax.experimental.pallas import tpu_sc as plsc

NEG_BIG = -3.0e38
_SC_PARAMS = pltpu.CompilerParams(use_tc_tiling_on_sc=False,
                                  needs_layout_passes=False)
CH = 128
NW = 32
NSUB = 16


def _tree_sum(vs):
    vs = list(vs)
    while len(vs) > 1:
        nxt = [vs[i] + vs[i + 1] for i in range(0, len(vs) - 1, 2)]
        if len(vs) % 2:
            nxt.append(vs[-1])
        vs = nxt
    return vs[0]


def _build_proj(NP, D, H):
    TN = 512
    grid = (NP // TN,)

    def body(x_ref, w_ref, b_ref, qk_ref, v0_ref, v1_ref, s_ref):
        res = jnp.dot(x_ref[...], w_ref[...],
                      preferred_element_type=jnp.float32) + b_ref[...]
        qk_ref[0] = res[:, 0:H]
        qk_ref[1] = res[:, H:2 * H]
        qk_ref[2] = res[:, 0:H]
        qk_ref[3] = res[:, H:2 * H]
        v0_ref[...] = res[:, 2 * H:2 * H + H // 2]
        v1_ref[...] = res[:, 2 * H + H // 2:3 * H]
        s_ref[0] = res[:, 3 * H:3 * H + H // 2]
        s_ref[1] = res[:, 3 * H + H // 2:4 * H]

    return pl.pallas_call(
        body,
        grid=grid,
        in_specs=[
            pl.BlockSpec((TN, D), lambda i: (i, 0)),
            pl.BlockSpec((D, 4 * H), lambda i: (0, 0)),
            pl.BlockSpec((1, 4 * H), lambda i: (0, 0)),
        ],
        out_specs=[
            pl.BlockSpec((4, TN, H), lambda i: (0, i, 0)),
            pl.BlockSpec((TN, H // 2), lambda i: (i, 0)),
            pl.BlockSpec((TN, H // 2), lambda i: (i, 0)),
            pl.BlockSpec((2, TN, H // 2), lambda i: (0, i, 0)),
        ],
        out_shape=[
            jax.ShapeDtypeStruct((4, NP, H), jnp.float32),
            jax.ShapeDtypeStruct((NP, H // 2), jnp.float32),
            jax.ShapeDtypeStruct((NP, H // 2), jnp.float32),
            jax.ShapeDtypeStruct((2, NP, H // 2), jnp.float32),
        ],
    )


def _build_k1(NP, Ep, H, CH1, CHW1, EPW):
    inv_sqrt_h = 1.0 / (H ** 0.5)
    mesh = plsc.VectorSubcoreMesh(core_axis_name="c", subcore_axis_name="s")

    @functools.partial(
        pl.kernel,
        out_type=(jax.ShapeDtypeStruct((NW, EPW), jnp.float32),
                  jax.ShapeDtypeStruct((NW, 16), jnp.float32)),
        mesh=mesh,
        compiler_params=_SC_PARAMS,
        scratch_types=[
            pltpu.VMEM((CHW1, CH1), jnp.int32),
            pltpu.VMEM((CHW1, CH1), jnp.int32),
            pltpu.VMEM((CH1, H), jnp.float32),
            pltpu.VMEM((CH1, H), jnp.float32),
            pltpu.VMEM((CH1, H), jnp.float32),
            pltpu.VMEM((CH1, H), jnp.float32),
            pltpu.VMEM((EPW,), jnp.float32),
            pltpu.VMEM((16, 16), jnp.float32),
            pltpu.VMEM((16,), jnp.float32),
            pltpu.SemaphoreType.DMA,
            pltpu.SemaphoreType.DMA,
            pltpu.SemaphoreType.DMA,
            pltpu.SemaphoreType.DMA,
        ],
    )
    def k1(qk_hbm, dst_hbm, src_hbm, alpha_hbm, mx_hbm,
           dst2d, src2d, qr0, kr0, qr1, kr1, alphabig, tbuf, mxbuf,
           sq0, sk0, sq1, sk1):
        c = lax.axis_index("c")
        s = lax.axis_index("s")
        wid = s * 2 + c
        pltpu.sync_copy(dst_hbm.at[wid], dst2d)
        pltpu.sync_copy(src_hbm.at[wid], src2d)
        iota = jnp.arange(16, dtype=jnp.int32)
        inv = jnp.float32(inv_sqrt_h)
        qoff = 2 * c * NP
        koff = qoff + NP

        def offbody(t, carry):
            for j in range(CH1 // 16):
                sl = pl.ds(j * 16, 16)
                dst2d[t, sl] = dst2d[t, sl] + qoff
                src2d[t, sl] = src2d[t, sl] + koff
            return carry

        lax.fori_loop(0, CHW1, offbody, jnp.int32(0))
        bufs = ((qr0, kr0, sq0, sk0), (qr1, kr1, sq1, sk1))

        def issue(t, b):
            qr, kr, sq, sk = bufs[b]
            pltpu.async_copy(qk_hbm.at[dst2d.at[t]], qr, sq)
            pltpu.async_copy(qk_hbm.at[src2d.at[t]], kr, sk)

        def wait(t, b):
            qr, kr, sq, sk = bufs[b]
            pltpu.make_async_copy(qk_hbm.at[dst2d.at[t]], qr, sq).wait()
            pltpu.make_async_copy(qk_hbm.at[src2d.at[t]], kr, sk).wait()

        def compute(t, b, mxv):
            qr, kr, _, _ = bufs[b]
            for grp in range(CH1 // 16):

                def ebody(e, carry):
                    r = grp * 16 + e
                    ps = [qr[r, pl.ds(j * 16, 16)] * kr[r, pl.ds(j * 16, 16)]
                          for j in range(H // 16)]
                    tbuf[e, pl.ds(0, 16)] = _tree_sum(ps)
                    return carry

                lax.fori_loop(0, 16, ebody, jnp.int32(0))
                cols = [plsc.load_gather(tbuf, [iota, jnp.full((16,), j, jnp.int32)])
                        for j in range(16)]
                a16 = _tree_sum(cols) * inv
                alphabig[pl.ds(t * CH1 + grp * 16, 16)] = a16
                mxv = jnp.maximum(mxv, a16)
            return mxv

        issue(0, 0)

        def pair_body(p, mxv):
            t0 = p * 2
            t1 = t0 + 1
            issue(t1, 1)
            wait(t0, 0)
            mxv = compute(t0, 0, mxv)

            @pl.when(t0 + 2 < CHW1)
            def _():
                issue(t0 + 2, 0)

            wait(t1, 1)
            mxv = compute(t1, 1, mxv)
            return mxv

        mxv = lax.fori_loop(0, CHW1 // 2, pair_body,
                            jnp.full((16,), NEG_BIG, jnp.float32))
        mxbuf[...] = mxv
        pltpu.sync_copy(alphabig, alpha_hbm.at[wid])
        pltpu.sync_copy(mxbuf, mx_hbm.at[wid])

    return k1


def _build_k2(ND, SLICE, CHW, EPW):
    mesh = plsc.VectorSubcoreMesh(core_axis_name="c", subcore_axis_name="s")

    @functools.partial(
        pl.kernel,
        out_type=(jax.ShapeDtypeStruct((NW, EPW), jnp.float32),
                  jax.ShapeDtypeStruct((2, ND), jnp.float32)),
        mesh=mesh,
        compiler_params=_SC_PARAMS,
        scratch_types=[
            pltpu.VMEM((NW, 16), jnp.float32),
            pltpu.VMEM((CHW, CH), jnp.int32),
            pltpu.VMEM((EPW,), jnp.float32),
            pltpu.VMEM((EPW,), jnp.float32),
            pltpu.VMEM_SHARED((ND,), jnp.float32),
        ],
    )
    def k2(alpha_hbm, dst_hbm, mx_hbm, znd_hbm, ex_hbm, den_hbm,
           mxbuf, dst2d, alphabig, exbig, denom_sp):
        c = lax.axis_index("c")
        s = lax.axis_index("s")
        wid = s * 2 + c
        pltpu.sync_copy(mx_hbm, mxbuf)

        def mbody(i, m):
            return jnp.maximum(m, mxbuf[i])

        m = lax.fori_loop(0, NW, mbody, jnp.full((16,), NEG_BIG, jnp.float32))
        cmax = jnp.max(m)
        cvec = jnp.full((16,), cmax)
        pltpu.sync_copy(znd_hbm.at[pl.ds(s * SLICE, SLICE)],
                        denom_sp.at[pl.ds(s * SLICE, SLICE)])
        pltpu.sync_copy(alpha_hbm.at[wid], alphabig)
        pltpu.sync_copy(dst_hbm.at[wid], dst2d)

        def gbody(g, carry):
            sl = pl.ds(g * 16, 16)
            exbig[sl] = jnp.exp(alphabig[sl] - cvec)
            return carry

        lax.fori_loop(0, EPW // 16, gbody, jnp.int32(0))
        pltpu.sync_copy(exbig, ex_hbm.at[wid])
        plsc.subcore_barrier()

        def sbody(t, carry):
            pltpu.sync_copy(exbig.at[pl.ds(t * CH, CH)],
                            denom_sp.at[dst2d.at[t]], add=True)
            return carry

        lax.fori_loop(0, CHW, sbody, jnp.int32(0))
        plsc.subcore_barrier()
        pltpu.sync_copy(denom_sp.at[pl.ds(s * SLICE, SLICE)],
                        den_hbm.at[c, pl.ds(s * SLICE, SLICE)])

    return k2


def _build_k3(N, NP, ND, SLICE, HH, CHS, ESUB):
    last_rows = N - (NSUB - 1) * SLICE
    GC = 8
    NG = CHS // GC
    DB = ND // 4
    mesh = plsc.VectorSubcoreMesh(core_axis_name="c", subcore_axis_name="s")

    @functools.partial(
        pl.kernel,
        out_type=jax.ShapeDtypeStruct((2, NSUB, HH), jnp.float32),
        mesh=mesh,
        compiler_params=_SC_PARAMS,
        scratch_types=[
            pltpu.VMEM((ND,), jnp.float32),
            pltpu.VMEM((DB,), jnp.float32),
            pltpu.VMEM((GC, CH), jnp.int32),
            pltpu.VMEM((GC, CH), jnp.int32),
            pltpu.VMEM((GC * CH,), jnp.float32),
            pltpu.VMEM((CH,), jnp.float32),
            pltpu.VMEM((CH, HH), jnp.float32),
            pltpu.VMEM((CH, HH), jnp.float32),
            pltpu.VMEM_SHARED((ND, HH), jnp.float32),
            pltpu.SemaphoreType.DMA,
            pltpu.SemaphoreType.DMA,
        ],
    )
    def k3(vcat_hbm, ex_hbm, dst_hbm, src_hbm, den_hbm, zagg_hbm, sxc_hbm,
           pout_hbm,
           rdenom, dbuf, dstg, srcg, exg, wbuf, vr0, vr1, agg_sp, sg0, sg1):
        c = lax.axis_index("c")
        s = lax.axis_index("s")
        srcoff = c * NP
        pltpu.sync_copy(den_hbm.at[0], rdenom)
        for blk in range(4):
            pltpu.sync_copy(den_hbm.at[1, pl.ds(blk * DB, DB)], dbuf)

            def rbody(i, carry, _blk=blk):
                sl16 = pl.ds(_blk * DB + i * 16, 16)
                rdenom[sl16] = 1.0 / (rdenom[sl16] + dbuf[pl.ds(i * 16, 16)]
                                      + jnp.float32(1e-16))
                return carry

            lax.fori_loop(0, DB // 16, rbody, jnp.int32(0))
        pltpu.sync_copy(zagg_hbm, agg_sp.at[pl.ds(s * SLICE, SLICE)])
        plsc.subcore_barrier()
        bufs = ((vr0, sg0), (vr1, sg1))

        def issue(t, b):
            vr, sg = bufs[b]
            pltpu.async_copy(vcat_hbm.at[srcg.at[t]], vr, sg)

        def wait(t, b):
            vr, sg = bufs[b]
            pltpu.make_async_copy(vcat_hbm.at[srcg.at[t]], vr, sg).wait()

        def scale_scatter(t, b):
            vr, _ = bufs[b]
            for grp in range(CH // 16):
                sl = pl.ds(grp * 16, 16)
                d16 = dstg[t, sl]
                rd = plsc.load_gather(rdenom, [d16])
                wbuf[sl] = exg[pl.ds(t * CH + grp * 16, 16)] * rd

            def ebody(e, carry3):
                wsp = plsc.load_gather(wbuf, [jnp.full((16,), e, jnp.int32)])
                for cb in range(HH // 16):
                    slc = pl.ds(cb * 16, 16)
                    vr[e, slc] = vr[e, slc] * wsp
                return carry3

            lax.fori_loop(0, CH, ebody, jnp.int32(0), unroll=2)
            pltpu.sync_copy(vr, agg_sp.at[dstg.at[t]], add=True)

        def group_body(gi, carry):
            pltpu.sync_copy(dst_hbm.at[s, pl.ds(gi * GC, GC)], dstg)
            pltpu.sync_copy(src_hbm.at[s, pl.ds(gi * GC, GC)], srcg)
            pltpu.sync_copy(ex_hbm.at[s, pl.ds(gi * GC * CH, GC * CH)], exg)

            def offbody(t, carry2):
                for j in range(CH // 16):
                    sl = pl.ds(j * 16, 16)
                    srcg[t, sl] = srcg[t, sl] + srcoff
                return carry2

            lax.fori_loop(0, GC, offbody, jnp.int32(0))
            issue(0, 0)

            def pair_body(p, carry2):
                t0 = p * 2
                t1 = t0 + 1
                issue(t1, 1)
                wait(t0, 0)
                scale_scatter(t0, 0)

                @pl.when(t0 + 2 < GC)
                def _():
                    issue(t0 + 2, 0)

                wait(t1, 1)
                scale_scatter(t1, 1)
                return carry2

            lax.fori_loop(0, GC // 2, pair_body, jnp.int32(0))
            return carry

        lax.fori_loop(0, NG, group_body, jnp.int32(0))
        plsc.subcore_barrier()
        m8 = [jnp.full((16,), NEG_BIG, jnp.float32) for _ in range(HH // 16)]
        for blk in range(SLICE // CH):
            base = s * SLICE + blk * CH
            pltpu.sync_copy(agg_sp.at[pl.ds(base, CH)], vr1)
            pltpu.sync_copy(sxc_hbm.at[pl.ds(c * NP + base, CH)], vr0)

            def pbody(r, carry):
                row_ok = base + r < N
                out = []
                for j in range(HH // 16):
                    slj = pl.ds(j * 16, 16)
                    hj = vr1[r, slj] + vr0[r, slj]
                    hj = jnp.where(row_ok, hj, jnp.full((16,), NEG_BIG, jnp.float32))
                    out.append(jnp.maximum(carry[j], hj))
                return tuple(out)

            m8 = lax.fori_loop(0, CH, pbody, tuple(m8))
            m8 = list(m8)
        for j in range(HH // 16):
            wbuf[pl.ds(j * 16, 16)] = m8[j]
        pltpu.sync_copy(wbuf, pout_hbm.at[c, s])

    return k3


def _build_mlp1(B, G, H, P, M):
    TK = 1000
    nk = G // TK
    grid = (nk,)

    def body(ct_ref, wm1_ref, pert_ref, wp_ref, bp_ref, w1b_ref, w1c_ref,
             pooled_ref, bm1_ref, out_ref):
        i = pl.program_id(0)

        @pl.when(i == 0)
        def _():
            out_ref[...] = jnp.zeros_like(out_ref)

        out_ref[...] += jax.lax.dot_general(
            ct_ref[...], wm1_ref[...], (((0,), (0,)), ((), ())),
            preferred_element_type=jnp.float32)

        @pl.when(i == nk - 1)
        def _():
            emb = jnp.dot(pert_ref[...], wp_ref[...],
                          preferred_element_type=jnp.float32) + bp_ref[...]
            acc2 = jnp.dot(emb, w1b_ref[...], preferred_element_type=jnp.float32)
            t = jnp.dot(pooled_ref[...], w1c_ref[...],
                        preferred_element_type=jnp.float32)
            z = out_ref[...] + acc2 + t + bm1_ref[...]
            out_ref[...] = jax.nn.softplus(z)

    return pl.pallas_call(
        body,
        grid=grid,
        in_specs=[
            pl.BlockSpec((TK, B), lambda i: (i, 0)),
            pl.BlockSpec((TK, M), lambda i: (i, 0)),
            pl.BlockSpec((B, P), lambda i: (0, 0)),
            pl.BlockSpec((P, P), lambda i: (0, 0)),
            pl.BlockSpec((1, P), lambda i: (0, 0)),
            pl.BlockSpec((P, M), lambda i: (0, 0)),
            pl.BlockSpec((H, M), lambda i: (0, 0)),
            pl.BlockSpec((1, H), lambda i: (0, 0)),
            pl.BlockSpec((1, M), lambda i: (0, 0)),
        ],
        out_specs=pl.BlockSpec((B, M), lambda i: (0, 0)),
        out_shape=jax.ShapeDtypeStruct((B, M), jnp.float32),
    )


def _build_mlp2(B, G, M):
    TKM = 256
    nk = M // TKM
    grid = (nk,)

    def body(h1_ref, w2_ref, b2_ref, out_ref):
        i = pl.program_id(0)

        @pl.when(i == 0)
        def _():
            out_ref[...] = jnp.zeros_like(out_ref)

        out_ref[...] += jnp.dot(h1_ref[...], w2_ref[...],
                                preferred_element_type=jnp.float32)

        @pl.when(i == nk - 1)
        def _():
            out_ref[...] += b2_ref[...]

    return pl.pallas_call(
        body,
        grid=grid,
        in_specs=[
            pl.BlockSpec((B, TKM), lambda i: (0, i)),
            pl.BlockSpec((TKM, G), lambda i: (i, 0)),
            pl.BlockSpec((1, G), lambda i: (0, 0)),
        ],
        out_specs=pl.BlockSpec((B, G), lambda i: (0, 0)),
        out_shape=jax.ShapeDtypeStruct((B, G), jnp.float32),
    )


def kernel(x, edge_index, ctrl, pert, pos, Wq, bq, Wk, bk, Wv, bv,
           Wskip, bskip, W1, b1, Wp, bp, Wm1, bm1, Wm2, bm2):
    N, D = x.shape
    E = edge_index.shape[1]
    H = Wq.shape[1]
    B, G = ctrl.shape
    P = pert.shape[1]
    M = Wm1.shape[1]
    HH = H // 2
    NP = ((N + NW * 16 - 1) // (NW * 16)) * (NW * 16)
    ND = NP
    SLICE = ND // NSUB
    Ep = ((E + NW * CH - 1) // (NW * CH)) * (NW * CH)
    EPW = Ep // NW
    CHW = EPW // CH
    ESUB = Ep // NSUB
    CHS = ESUB // CH

    xp = jnp.pad(x, ((0, NP - N), (0, 0)))
    pad_ids = N + (jnp.arange(Ep - E, dtype=jnp.int32) % (NP - N))
    src = jnp.concatenate([edge_index[0], pad_ids])
    dst = jnp.concatenate([edge_index[1], pad_ids])
    CH1 = 64
    CHW1 = EPW // CH1
    dstw = dst.reshape(NW, CHW, CH)
    srcw = src.reshape(NW, CHW, CH)
    dstw1 = dst.reshape(NW, CHW1, CH1)
    srcw1 = src.reshape(NW, CHW1, CH1)
    dsts = dst.reshape(NSUB, CHS, CH)
    srcs = src.reshape(NSUB, CHS, CH)

    wbig = jnp.concatenate([Wq, Wk, Wv, Wskip + W1], axis=1)
    bbig = jnp.concatenate([bq, bk, bv, bskip + b1])[None, :]
    qk4, v0, v1, sxc = _build_proj(NP, D, H)(xp, wbig, bbig)
    sxcat = sxc.reshape(2 * NP, HH)
    qkflat = qk4.reshape(4 * NP, H)

    alpha, mx = _build_k1(NP, Ep, H, CH1, CHW1, EPW)(qkflat, dstw1, srcw1)
    znd = jnp.zeros((ND,), jnp.float32)
    ex, den2 = _build_k2(ND, SLICE, CHW, EPW)(alpha, dstw, mx, znd)
    vcat = jnp.concatenate([v0, v1], axis=0)
    zagg = jnp.zeros((SLICE, HH), jnp.float32)
    exs = ex.reshape(NSUB, ESUB)
    pout = _build_k3(N, NP, ND, SLICE, HH, CHS, ESUB)(
        vcat, exs, dsts, srcs, den2, zagg, sxcat)

    pooled = jnp.max(pout, axis=1).reshape(1, H)

    ctrl_t = ctrl.T
    w1c = Wm1[G:G + H]
    w1b = Wm1[G + H:]
    h1 = _build_mlp1(B, G, H, P, M)(ctrl_t, Wm1, pert, Wp, bp[None], w1b,
                                    w1c, pooled, bm1[None])
    out = _build_mlp2(B, G, M)(h1, Wm2, bm2[None])
    return out


# speedup vs baseline: 1.8325x; 1.0341x over previous
import functools

import jax
import jax.numpy as jnp
from jax import lax
from jax.experimental import pallas as pl
from jax.experimental.pallas import tpu as pltpu
from jax.experimental.pallas import tpu_sc as plsc

NEG_BIG = -3.0e38
_SC_PARAMS = pltpu.CompilerParams(use_tc_tiling_on_sc=False,
                                  needs_layout_passes=False)
CH = 128
NW = 32
NSUB = 16


def _tree_sum(vs):
    vs = list(vs)
    while len(vs) > 1:
        nxt = [vs[i] + vs[i + 1] for i in range(0, len(vs) - 1, 2)]
        if len(vs) % 2:
            nxt.append(vs[-1])
        vs = nxt
    return vs[0]


def _build_proj(NP, D, H):
    TN = 512
    grid = (NP // TN,)

    def body(x_ref, w_ref, b_ref, qk_ref, v0_ref, v1_ref, s_ref):
        res = jnp.dot(x_ref[...], w_ref[...],
                      preferred_element_type=jnp.float32) + b_ref[...]
        qk_ref[0] = res[:, 0:H]
        qk_ref[1] = res[:, H:2 * H]
        v0_ref[...] = res[:, 2 * H:2 * H + H // 2]
        v1_ref[...] = res[:, 2 * H + H // 2:3 * H]
        s_ref[0] = res[:, 3 * H:3 * H + H // 2]
        s_ref[1] = res[:, 3 * H + H // 2:4 * H]

    return pl.pallas_call(
        body,
        grid=grid,
        in_specs=[
            pl.BlockSpec((TN, D), lambda i: (i, 0)),
            pl.BlockSpec((D, 4 * H), lambda i: (0, 0)),
            pl.BlockSpec((1, 4 * H), lambda i: (0, 0)),
        ],
        out_specs=[
            pl.BlockSpec((2, TN, H), lambda i: (0, i, 0)),
            pl.BlockSpec((TN, H // 2), lambda i: (i, 0)),
            pl.BlockSpec((TN, H // 2), lambda i: (i, 0)),
            pl.BlockSpec((2, TN, H // 2), lambda i: (0, i, 0)),
        ],
        out_shape=[
            jax.ShapeDtypeStruct((2, NP, H), jnp.float32),
            jax.ShapeDtypeStruct((NP, H // 2), jnp.float32),
            jax.ShapeDtypeStruct((NP, H // 2), jnp.float32),
            jax.ShapeDtypeStruct((2, NP, H // 2), jnp.float32),
        ],
    )


def _build_k1(NP, Ep, H, CH1, CHW1, EPW):
    inv_sqrt_h = 1.0 / (H ** 0.5)
    mesh = plsc.VectorSubcoreMesh(core_axis_name="c", subcore_axis_name="s")

    @functools.partial(
        pl.kernel,
        out_type=(jax.ShapeDtypeStruct((NW, EPW), jnp.float32),
                  jax.ShapeDtypeStruct((NW, 16), jnp.float32)),
        mesh=mesh,
        compiler_params=_SC_PARAMS,
        scratch_types=[
            pltpu.VMEM((CHW1, CH1), jnp.int32),
            pltpu.VMEM((CHW1, CH1), jnp.int32),
            pltpu.VMEM((CH1, H), jnp.float32),
            pltpu.VMEM((CH1, H), jnp.float32),
            pltpu.VMEM((CH1, H), jnp.float32),
            pltpu.VMEM((CH1, H), jnp.float32),
            pltpu.VMEM((EPW,), jnp.float32),
            pltpu.VMEM((16, 16), jnp.float32),
            pltpu.VMEM((16,), jnp.float32),
            pltpu.SemaphoreType.DMA,
            pltpu.SemaphoreType.DMA,
            pltpu.SemaphoreType.DMA,
            pltpu.SemaphoreType.DMA,
        ],
    )
    def k1(qk_hbm, dst_hbm, src_hbm, alpha_hbm, mx_hbm,
           dst2d, src2d, qr0, kr0, qr1, kr1, alphabig, tbuf, mxbuf,
           sq0, sk0, sq1, sk1):
        c = lax.axis_index("c")
        s = lax.axis_index("s")
        wid = s * 2 + c
        pltpu.sync_copy(dst_hbm.at[wid], dst2d)
        pltpu.sync_copy(src_hbm.at[wid], src2d)
        iota = jnp.arange(16, dtype=jnp.int32)
        inv = jnp.float32(inv_sqrt_h)
        koff = NP

        def offbody(t, carry):
            for j in range(CH1 // 16):
                sl = pl.ds(j * 16, 16)
                src2d[t, sl] = src2d[t, sl] + koff
            return carry

        lax.fori_loop(0, CHW1, offbody, jnp.int32(0))
        bufs = ((qr0, kr0, sq0, sk0), (qr1, kr1, sq1, sk1))

        def issue(t, b):
            qr, kr, sq, sk = bufs[b]
            pltpu.async_copy(qk_hbm.at[dst2d.at[t]], qr, sq)
            pltpu.async_copy(qk_hbm.at[src2d.at[t]], kr, sk)

        def wait(t, b):
            qr, kr, sq, sk = bufs[b]
            pltpu.make_async_copy(qk_hbm.at[dst2d.at[t]], qr, sq).wait()
            pltpu.make_async_copy(qk_hbm.at[src2d.at[t]], kr, sk).wait()

        def compute(t, b, mxv):
            qr, kr, _, _ = bufs[b]
            for grp in range(CH1 // 16):

                def ebody(e, carry):
                    r = grp * 16 + e
                    ps = [qr[r, pl.ds(j * 16, 16)] * kr[r, pl.ds(j * 16, 16)]
                          for j in range(H // 16)]
                    tbuf[e, pl.ds(0, 16)] = _tree_sum(ps)
                    return carry

                lax.fori_loop(0, 16, ebody, jnp.int32(0))
                cols = [plsc.load_gather(tbuf, [iota, jnp.full((16,), j, jnp.int32)])
                        for j in range(16)]
                a16 = _tree_sum(cols) * inv
                alphabig[pl.ds(t * CH1 + grp * 16, 16)] = a16
                mxv = jnp.maximum(mxv, a16)
            return mxv

        issue(0, 0)

        def pair_body(p, mxv):
            t0 = p * 2
            t1 = t0 + 1
            issue(t1, 1)
            wait(t0, 0)
            mxv = compute(t0, 0, mxv)

            @pl.when(t0 + 2 < CHW1)
            def _():
                issue(t0 + 2, 0)

            wait(t1, 1)
            mxv = compute(t1, 1, mxv)
            return mxv

        mxv = lax.fori_loop(0, CHW1 // 2, pair_body,
                            jnp.full((16,), NEG_BIG, jnp.float32))
        mxbuf[...] = mxv
        pltpu.sync_copy(alphabig, alpha_hbm.at[wid])
        pltpu.sync_copy(mxbuf, mx_hbm.at[wid])

    return k1


def _build_k2(ND, SLICE, CHW, EPW):
    mesh = plsc.VectorSubcoreMesh(core_axis_name="c", subcore_axis_name="s")

    @functools.partial(
        pl.kernel,
        out_type=(jax.ShapeDtypeStruct((NW, EPW), jnp.float32),
                  jax.ShapeDtypeStruct((2, ND), jnp.float32)),
        mesh=mesh,
        compiler_params=_SC_PARAMS,
        scratch_types=[
            pltpu.VMEM((NW, 16), jnp.float32),
            pltpu.VMEM((CHW, CH), jnp.int32),
            pltpu.VMEM((EPW,), jnp.float32),
            pltpu.VMEM((EPW,), jnp.float32),
            pltpu.VMEM_SHARED((ND,), jnp.float32),
        ],
    )
    def k2(alpha_hbm, dst_hbm, mx_hbm, znd_hbm, ex_hbm, den_hbm,
           mxbuf, dst2d, alphabig, exbig, denom_sp):
        c = lax.axis_index("c")
        s = lax.axis_index("s")
        wid = s * 2 + c
        pltpu.sync_copy(mx_hbm, mxbuf)

        def mbody(i, m):
            return jnp.maximum(m, mxbuf[i])

        m = lax.fori_loop(0, NW, mbody, jnp.full((16,), NEG_BIG, jnp.float32))
        cmax = jnp.max(m)
        cvec = jnp.full((16,), cmax)
        pltpu.sync_copy(znd_hbm.at[pl.ds(s * SLICE, SLICE)],
                        denom_sp.at[pl.ds(s * SLICE, SLICE)])
        pltpu.sync_copy(alpha_hbm.at[wid], alphabig)
        pltpu.sync_copy(dst_hbm.at[wid], dst2d)

        def gbody(g, carry):
            sl = pl.ds(g * 16, 16)
            exbig[sl] = jnp.exp(alphabig[sl] - cvec)
            return carry

        lax.fori_loop(0, EPW // 16, gbody, jnp.int32(0))
        pltpu.sync_copy(exbig, ex_hbm.at[wid])
        plsc.subcore_barrier()

        def sbody(t, carry):
            pltpu.sync_copy(exbig.at[pl.ds(t * CH, CH)],
                            denom_sp.at[dst2d.at[t]], add=True)
            return carry

        lax.fori_loop(0, CHW, sbody, jnp.int32(0))
        plsc.subcore_barrier()
        pltpu.sync_copy(denom_sp.at[pl.ds(s * SLICE, SLICE)],
                        den_hbm.at[c, pl.ds(s * SLICE, SLICE)])

    return k2


def _build_k3(N, NP, ND, SLICE, HH, CHS, ESUB):
    last_rows = N - (NSUB - 1) * SLICE
    GC = 8
    NG = CHS // GC
    DB = ND // 4
    mesh = plsc.VectorSubcoreMesh(core_axis_name="c", subcore_axis_name="s")

    @functools.partial(
        pl.kernel,
        out_type=jax.ShapeDtypeStruct((2, NSUB, HH), jnp.float32),
        mesh=mesh,
        compiler_params=_SC_PARAMS,
        scratch_types=[
            pltpu.VMEM((ND,), jnp.float32),
            pltpu.VMEM((DB,), jnp.float32),
            pltpu.VMEM((GC, CH), jnp.int32),
            pltpu.VMEM((GC, CH), jnp.int32),
            pltpu.VMEM((GC * CH,), jnp.float32),
            pltpu.VMEM((CH,), jnp.float32),
            pltpu.VMEM((CH, HH), jnp.float32),
            pltpu.VMEM((CH, HH), jnp.float32),
            pltpu.VMEM_SHARED((ND, HH), jnp.float32),
            pltpu.SemaphoreType.DMA,
            pltpu.SemaphoreType.DMA,
        ],
    )
    def k3(vcat_hbm, ex_hbm, dst_hbm, src_hbm, den_hbm, zagg_hbm, sxc_hbm,
           pout_hbm,
           rdenom, dbuf, dstg, srcg, exg, wbuf, vr0, vr1, agg_sp, sg0, sg1):
        c = lax.axis_index("c")
        s = lax.axis_index("s")
        srcoff = c * NP
        pltpu.sync_copy(den_hbm.at[0], rdenom)
        for blk in range(4):
            pltpu.sync_copy(den_hbm.at[1, pl.ds(blk * DB, DB)], dbuf)

            def rbody(i, carry, _blk=blk):
                sl16 = pl.ds(_blk * DB + i * 16, 16)
                rdenom[sl16] = 1.0 / (rdenom[sl16] + dbuf[pl.ds(i * 16, 16)]
                                      + jnp.float32(1e-16))
                return carry

            lax.fori_loop(0, DB // 16, rbody, jnp.int32(0))
        pltpu.sync_copy(zagg_hbm, agg_sp.at[pl.ds(s * SLICE, SLICE)])
        plsc.subcore_barrier()
        bufs = ((vr0, sg0), (vr1, sg1))

        def issue(t, b):
            vr, sg = bufs[b]
            pltpu.async_copy(vcat_hbm.at[srcg.at[t]], vr, sg)

        def wait(t, b):
            vr, sg = bufs[b]
            pltpu.make_async_copy(vcat_hbm.at[srcg.at[t]], vr, sg).wait()

        def scale_scatter(t, b):
            vr, _ = bufs[b]
            for grp in range(CH // 16):
                sl = pl.ds(grp * 16, 16)
                d16 = dstg[t, sl]
                rd = plsc.load_gather(rdenom, [d16])
                wbuf[sl] = exg[pl.ds(t * CH + grp * 16, 16)] * rd

            def ebody(e, carry3):
                wsp = plsc.load_gather(wbuf, [jnp.full((16,), e, jnp.int32)])
                for cb in range(HH // 16):
                    slc = pl.ds(cb * 16, 16)
                    vr[e, slc] = vr[e, slc] * wsp
                return carry3

            lax.fori_loop(0, CH, ebody, jnp.int32(0), unroll=2)
            pltpu.sync_copy(vr, agg_sp.at[dstg.at[t]], add=True)

        def group_body(gi, carry):
            pltpu.sync_copy(dst_hbm.at[s, pl.ds(gi * GC, GC)], dstg)
            pltpu.sync_copy(src_hbm.at[s, pl.ds(gi * GC, GC)], srcg)
            pltpu.sync_copy(ex_hbm.at[s, pl.ds(gi * GC * CH, GC * CH)], exg)

            def offbody(t, carry2):
                for j in range(CH // 16):
                    sl = pl.ds(j * 16, 16)
                    srcg[t, sl] = srcg[t, sl] + srcoff
                return carry2

            lax.fori_loop(0, GC, offbody, jnp.int32(0))
            issue(0, 0)

            def pair_body(p, carry2):
                t0 = p * 2
                t1 = t0 + 1
                issue(t1, 1)
                wait(t0, 0)
                scale_scatter(t0, 0)

                @pl.when(t0 + 2 < GC)
                def _():
                    issue(t0 + 2, 0)

                wait(t1, 1)
                scale_scatter(t1, 1)
                return carry2

            lax.fori_loop(0, GC // 2, pair_body, jnp.int32(0))
            return carry

        lax.fori_loop(0, NG, group_body, jnp.int32(0))
        plsc.subcore_barrier()
        m8 = [jnp.full((16,), NEG_BIG, jnp.float32) for _ in range(HH // 16)]
        for blk in range(SLICE // CH):
            base = s * SLICE + blk * CH
            pltpu.sync_copy(agg_sp.at[pl.ds(base, CH)], vr1)
            pltpu.sync_copy(sxc_hbm.at[pl.ds(c * NP + base, CH)], vr0)

            def pbody(r, carry):
                row_ok = base + r < N
                out = []
                for j in range(HH // 16):
                    slj = pl.ds(j * 16, 16)
                    hj = vr1[r, slj] + vr0[r, slj]
                    hj = jnp.where(row_ok, hj, jnp.full((16,), NEG_BIG, jnp.float32))
                    out.append(jnp.maximum(carry[j], hj))
                return tuple(out)

            m8 = lax.fori_loop(0, CH, pbody, tuple(m8))
            m8 = list(m8)
        for j in range(HH // 16):
            wbuf[pl.ds(j * 16, 16)] = m8[j]
        pltpu.sync_copy(wbuf, pout_hbm.at[c, s])

    return k3


def _build_mlp1(B, G, H, P, M):
    TK = 1000
    nk = G // TK
    grid = (nk,)

    def body(ct_ref, wm1_ref, pert_ref, wp_ref, bp_ref, w1b_ref, w1c_ref,
             pooled_ref, bm1_ref, out_ref):
        i = pl.program_id(0)

        @pl.when(i == 0)
        def _():
            out_ref[...] = jnp.zeros_like(out_ref)

        out_ref[...] += jax.lax.dot_general(
            ct_ref[...], wm1_ref[...], (((0,), (0,)), ((), ())),
            preferred_element_type=jnp.float32)

        @pl.when(i == nk - 1)
        def _():
            emb = jnp.dot(pert_ref[...], wp_ref[...],
                          preferred_element_type=jnp.float32) + bp_ref[...]
            acc2 = jnp.dot(emb, w1b_ref[...], preferred_element_type=jnp.float32)
            t = jnp.dot(pooled_ref[...], w1c_ref[...],
                        preferred_element_type=jnp.float32)
            z = out_ref[...] + acc2 + t + bm1_ref[...]
            out_ref[...] = jax.nn.softplus(z)

    return pl.pallas_call(
        body,
        grid=grid,
        in_specs=[
            pl.BlockSpec((TK, B), lambda i: (i, 0)),
            pl.BlockSpec((TK, M), lambda i: (i, 0)),
            pl.BlockSpec((B, P), lambda i: (0, 0)),
            pl.BlockSpec((P, P), lambda i: (0, 0)),
            pl.BlockSpec((1, P), lambda i: (0, 0)),
            pl.BlockSpec((P, M), lambda i: (0, 0)),
            pl.BlockSpec((H, M), lambda i: (0, 0)),
            pl.BlockSpec((1, H), lambda i: (0, 0)),
            pl.BlockSpec((1, M), lambda i: (0, 0)),
        ],
        out_specs=pl.BlockSpec((B, M), lambda i: (0, 0)),
        out_shape=jax.ShapeDtypeStruct((B, M), jnp.float32),
    )


def _build_mlp2(B, G, M):
    TKM = 256
    nk = M // TKM
    grid = (nk,)

    def body(h1_ref, w2_ref, b2_ref, out_ref):
        i = pl.program_id(0)

        @pl.when(i == 0)
        def _():
            out_ref[...] = jnp.zeros_like(out_ref)

        out_ref[...] += jnp.dot(h1_ref[...], w2_ref[...],
                                preferred_element_type=jnp.float32)

        @pl.when(i == nk - 1)
        def _():
            out_ref[...] += b2_ref[...]

    return pl.pallas_call(
        body,
        grid=grid,
        in_specs=[
            pl.BlockSpec((B, TKM), lambda i: (0, i)),
            pl.BlockSpec((TKM, G), lambda i: (i, 0)),
            pl.BlockSpec((1, G), lambda i: (0, 0)),
        ],
        out_specs=pl.BlockSpec((B, G), lambda i: (0, 0)),
        out_shape=jax.ShapeDtypeStruct((B, G), jnp.float32),
    )


def kernel(x, edge_index, ctrl, pert, pos, Wq, bq, Wk, bk, Wv, bv,
           Wskip, bskip, W1, b1, Wp, bp, Wm1, bm1, Wm2, bm2):
    N, D = x.shape
    E = edge_index.shape[1]
    H = Wq.shape[1]
    B, G = ctrl.shape
    P = pert.shape[1]
    M = Wm1.shape[1]
    HH = H // 2
    NP = ((N + NW * 16 - 1) // (NW * 16)) * (NW * 16)
    ND = NP
    SLICE = ND // NSUB
    Ep = ((E + NW * CH - 1) // (NW * CH)) * (NW * CH)
    EPW = Ep // NW
    CHW = EPW // CH
    ESUB = Ep // NSUB
    CHS = ESUB // CH

    xp = jnp.pad(x, ((0, NP - N), (0, 0)))
    pad_ids = N + (jnp.arange(Ep - E, dtype=jnp.int32) % (NP - N))
    src = jnp.concatenate([edge_index[0], pad_ids])
    dst = jnp.concatenate([edge_index[1], pad_ids])
    CH1 = 64
    CHW1 = EPW // CH1
    dstw = dst.reshape(NW, CHW, CH)
    srcw = src.reshape(NW, CHW, CH)
    dstw1 = dst.reshape(NW, CHW1, CH1)
    srcw1 = src.reshape(NW, CHW1, CH1)
    dsts = dst.reshape(NSUB, CHS, CH)
    srcs = src.reshape(NSUB, CHS, CH)

    wbig = jnp.concatenate([Wq, Wk, Wv, Wskip + W1], axis=1)
    bbig = jnp.concatenate([bq, bk, bv, bskip + b1])[None, :]
    qk4, v0, v1, sxc = _build_proj(NP, D, H)(xp, wbig, bbig)
    sxcat = sxc.reshape(2 * NP, HH)
    qkflat = qk4.reshape(2 * NP, H)

    alpha, mx = _build_k1(NP, Ep, H, CH1, CHW1, EPW)(qkflat, dstw1, srcw1)
    znd = jnp.zeros((ND,), jnp.float32)
    ex, den2 = _build_k2(ND, SLICE, CHW, EPW)(alpha, dstw, mx, znd)
    vcat = jnp.concatenate([v0, v1], axis=0)
    zagg = jnp.zeros((SLICE, HH), jnp.float32)
    exs = ex.reshape(NSUB, ESUB)
    pout = _build_k3(N, NP, ND, SLICE, HH, CHS, ESUB)(
        vcat, exs, dsts, srcs, den2, zagg, sxcat)

    pooled = jnp.max(pout, axis=1).reshape(1, H)

    ctrl_t = ctrl.T
    w1c = Wm1[G:G + H]
    w1b = Wm1[G + H:]
    h1 = _build_mlp1(B, G, H, P, M)(ctrl_t, Wm1, pert, Wp, bp[None], w1b,
                                    w1c, pooled, bm1[None])
    out = _build_mlp2(B, G, M)(h1, Wm2, bm2[None])
    return out


# speedup vs baseline: 1.8461x; 1.0074x over previous
import functools

import jax
import jax.numpy as jnp
from jax import lax
from jax.experimental import pallas as pl
from jax.experimental.pallas import tpu as pltpu
from jax.experimental.pallas import tpu_sc as plsc

NEG_BIG = -3.0e38
_SC_PARAMS = pltpu.CompilerParams(use_tc_tiling_on_sc=False,
                                  needs_layout_passes=False)
CH = 128
NW = 32
NSUB = 16


def _tree_sum(vs):
    vs = list(vs)
    while len(vs) > 1:
        nxt = [vs[i] + vs[i + 1] for i in range(0, len(vs) - 1, 2)]
        if len(vs) % 2:
            nxt.append(vs[-1])
        vs = nxt
    return vs[0]


def _build_proj(NP, D, H):
    TN = 512
    grid = (NP // TN,)

    def body(x_ref, w_ref, b_ref, qk_ref, v0_ref, v1_ref, s_ref):
        res = jnp.dot(x_ref[...], w_ref[...],
                      preferred_element_type=jnp.float32) + b_ref[...]
        qk_ref[0] = res[:, 0:H]
        qk_ref[1] = res[:, H:2 * H]
        v0_ref[...] = res[:, 2 * H:2 * H + H // 2]
        v1_ref[...] = res[:, 2 * H + H // 2:3 * H]
        s_ref[0] = res[:, 3 * H:3 * H + H // 2]
        s_ref[1] = res[:, 3 * H + H // 2:4 * H]

    return pl.pallas_call(
        body,
        grid=grid,
        in_specs=[
            pl.BlockSpec((TN, D), lambda i: (i, 0)),
            pl.BlockSpec((D, 4 * H), lambda i: (0, 0)),
            pl.BlockSpec((1, 4 * H), lambda i: (0, 0)),
        ],
        out_specs=[
            pl.BlockSpec((2, TN, H), lambda i: (0, i, 0)),
            pl.BlockSpec((TN, H // 2), lambda i: (i, 0)),
            pl.BlockSpec((TN, H // 2), lambda i: (i, 0)),
            pl.BlockSpec((2, TN, H // 2), lambda i: (0, i, 0)),
        ],
        out_shape=[
            jax.ShapeDtypeStruct((2, NP, H), jnp.float32),
            jax.ShapeDtypeStruct((NP, H // 2), jnp.float32),
            jax.ShapeDtypeStruct((NP, H // 2), jnp.float32),
            jax.ShapeDtypeStruct((2, NP, H // 2), jnp.float32),
        ],
    )


def _build_k1(NP, Ep, H, CH1, CHW1, EPW):
    inv_sqrt_h = 1.0 / (H ** 0.5)
    mesh = plsc.VectorSubcoreMesh(core_axis_name="c", subcore_axis_name="s")

    @functools.partial(
        pl.kernel,
        out_type=(jax.ShapeDtypeStruct((NW, EPW), jnp.float32),
                  jax.ShapeDtypeStruct((NW, 16), jnp.float32)),
        mesh=mesh,
        compiler_params=_SC_PARAMS,
        scratch_types=[
            pltpu.VMEM((CHW1, CH1), jnp.int32),
            pltpu.VMEM((CHW1, CH1), jnp.int32),
            pltpu.VMEM((CH1, H), jnp.float32),
            pltpu.VMEM((CH1, H), jnp.float32),
            pltpu.VMEM((CH1, H), jnp.float32),
            pltpu.VMEM((CH1, H), jnp.float32),
            pltpu.VMEM((EPW,), jnp.float32),
            pltpu.VMEM((16, 16), jnp.float32),
            pltpu.VMEM((16,), jnp.float32),
            pltpu.SemaphoreType.DMA,
            pltpu.SemaphoreType.DMA,
            pltpu.SemaphoreType.DMA,
            pltpu.SemaphoreType.DMA,
        ],
    )
    def k1(qk_hbm, dst_hbm, src_hbm, alpha_hbm, mx_hbm,
           dst2d, src2d, qr0, kr0, qr1, kr1, alphabig, tbuf, mxbuf,
           sq0, sk0, sq1, sk1):
        c = lax.axis_index("c")
        s = lax.axis_index("s")
        wid = s * 2 + c
        pltpu.sync_copy(dst_hbm.at[wid], dst2d)
        pltpu.sync_copy(src_hbm.at[wid], src2d)
        iota = jnp.arange(16, dtype=jnp.int32)
        inv = jnp.float32(inv_sqrt_h)
        koff = NP

        def offbody(t, carry):
            for j in range(CH1 // 16):
                sl = pl.ds(j * 16, 16)
                src2d[t, sl] = src2d[t, sl] + koff
            return carry

        lax.fori_loop(0, CHW1, offbody, jnp.int32(0))
        bufs = ((qr0, kr0, sq0, sk0), (qr1, kr1, sq1, sk1))

        def issue(t, b):
            qr, kr, sq, sk = bufs[b]
            pltpu.async_copy(qk_hbm.at[dst2d.at[t]], qr, sq)
            pltpu.async_copy(qk_hbm.at[src2d.at[t]], kr, sk)

        def wait(t, b):
            qr, kr, sq, sk = bufs[b]
            pltpu.make_async_copy(qk_hbm.at[dst2d.at[t]], qr, sq).wait()
            pltpu.make_async_copy(qk_hbm.at[src2d.at[t]], kr, sk).wait()

        def compute(t, b, mxv):
            qr, kr, _, _ = bufs[b]
            for grp in range(CH1 // 16):

                def ebody(e, carry):
                    r = grp * 16 + e
                    ps = [qr[r, pl.ds(j * 16, 16)] * kr[r, pl.ds(j * 16, 16)]
                          for j in range(H // 16)]
                    tbuf[e, pl.ds(0, 16)] = _tree_sum(ps)
                    return carry

                lax.fori_loop(0, 16, ebody, jnp.int32(0))
                cols = [plsc.load_gather(tbuf, [iota, jnp.full((16,), j, jnp.int32)])
                        for j in range(16)]
                a16 = _tree_sum(cols) * inv
                alphabig[pl.ds(t * CH1 + grp * 16, 16)] = a16
                mxv = jnp.maximum(mxv, a16)
            return mxv

        issue(0, 0)

        def pair_body(p, mxv):
            t0 = p * 2
            t1 = t0 + 1
            issue(t1, 1)
            wait(t0, 0)
            mxv = compute(t0, 0, mxv)

            @pl.when(t0 + 2 < CHW1)
            def _():
                issue(t0 + 2, 0)

            wait(t1, 1)
            mxv = compute(t1, 1, mxv)
            return mxv

        mxv = lax.fori_loop(0, CHW1 // 2, pair_body,
                            jnp.full((16,), NEG_BIG, jnp.float32))
        mxbuf[...] = mxv
        pltpu.sync_copy(alphabig, alpha_hbm.at[wid])
        pltpu.sync_copy(mxbuf, mx_hbm.at[wid])

    return k1


def _build_k2(ND, SLICE, CHW, EPW):
    mesh = plsc.VectorSubcoreMesh(core_axis_name="c", subcore_axis_name="s")

    @functools.partial(
        pl.kernel,
        out_type=(jax.ShapeDtypeStruct((NW, EPW), jnp.float32),
                  jax.ShapeDtypeStruct((2, ND), jnp.float32)),
        mesh=mesh,
        compiler_params=_SC_PARAMS,
        scratch_types=[
            pltpu.VMEM((NW, 16), jnp.float32),
            pltpu.VMEM((CHW, CH), jnp.int32),
            pltpu.VMEM((EPW,), jnp.float32),
            pltpu.VMEM((EPW,), jnp.float32),
            pltpu.VMEM_SHARED((ND,), jnp.float32),
        ],
    )
    def k2(alpha_hbm, dst_hbm, mx_hbm, znd_hbm, ex_hbm, den_hbm,
           mxbuf, dst2d, alphabig, exbig, denom_sp):
        c = lax.axis_index("c")
        s = lax.axis_index("s")
        wid = s * 2 + c
        pltpu.sync_copy(mx_hbm, mxbuf)

        def mbody(i, m):
            return jnp.maximum(m, mxbuf[i])

        m = lax.fori_loop(0, NW, mbody, jnp.full((16,), NEG_BIG, jnp.float32))
        cmax = jnp.max(m)
        cvec = jnp.full((16,), cmax)
        pltpu.sync_copy(znd_hbm.at[pl.ds(s * SLICE, SLICE)],
                        denom_sp.at[pl.ds(s * SLICE, SLICE)])
        pltpu.sync_copy(alpha_hbm.at[wid], alphabig)
        pltpu.sync_copy(dst_hbm.at[wid], dst2d)

        def gbody(g, carry):
            sl = pl.ds(g * 16, 16)
            exbig[sl] = jnp.exp(alphabig[sl] - cvec)
            return carry

        lax.fori_loop(0, EPW // 16, gbody, jnp.int32(0))
        pltpu.sync_copy(exbig, ex_hbm.at[wid])
        plsc.subcore_barrier()

        def sbody(t, carry):
            pltpu.sync_copy(exbig.at[pl.ds(t * CH, CH)],
                            denom_sp.at[dst2d.at[t]], add=True)
            return carry

        lax.fori_loop(0, CHW, sbody, jnp.int32(0))
        plsc.subcore_barrier()
        pltpu.sync_copy(denom_sp.at[pl.ds(s * SLICE, SLICE)],
                        den_hbm.at[c, pl.ds(s * SLICE, SLICE)])

    return k2


def _build_k3(N, NP, ND, SLICE, HH, CHS, ESUB):
    last_rows = N - (NSUB - 1) * SLICE
    GC = 8
    NG = CHS // GC
    DB = ND // 4
    mesh = plsc.VectorSubcoreMesh(core_axis_name="c", subcore_axis_name="s")

    @functools.partial(
        pl.kernel,
        out_type=jax.ShapeDtypeStruct((2, NSUB, HH), jnp.float32),
        mesh=mesh,
        compiler_params=_SC_PARAMS,
        scratch_types=[
            pltpu.VMEM((ND,), jnp.float32),
            pltpu.VMEM((DB,), jnp.float32),
            pltpu.VMEM((GC, CH), jnp.int32),
            pltpu.VMEM((GC, CH), jnp.int32),
            pltpu.VMEM((GC * CH,), jnp.float32),
            pltpu.VMEM((CH,), jnp.float32),
            pltpu.VMEM((CH, HH), jnp.float32),
            pltpu.VMEM((CH, HH), jnp.float32),
            pltpu.VMEM_SHARED((ND, HH), jnp.float32),
            pltpu.SemaphoreType.DMA,
            pltpu.SemaphoreType.DMA,
            pltpu.SemaphoreType.DMA,
            pltpu.SemaphoreType.DMA,
        ],
    )
    def k3(vcat_hbm, ex_hbm, dst_hbm, src_hbm, den_hbm, zagg_hbm, sxc_hbm,
           pout_hbm,
           rdenom, dbuf, dstg, srcg, exg, wbuf, vr0, vr1, agg_sp,
           sg0, sg1, ss0, ss1):
        c = lax.axis_index("c")
        s = lax.axis_index("s")
        srcoff = c * NP
        pltpu.sync_copy(den_hbm.at[0], rdenom)
        for blk in range(4):
            pltpu.sync_copy(den_hbm.at[1, pl.ds(blk * DB, DB)], dbuf)

            def rbody(i, carry, _blk=blk):
                sl16 = pl.ds(_blk * DB + i * 16, 16)
                rdenom[sl16] = 1.0 / (rdenom[sl16] + dbuf[pl.ds(i * 16, 16)]
                                      + jnp.float32(1e-16))
                return carry

            lax.fori_loop(0, DB // 16, rbody, jnp.int32(0))
        pltpu.sync_copy(zagg_hbm, agg_sp.at[pl.ds(s * SLICE, SLICE)])
        plsc.subcore_barrier()
        bufs = ((vr0, sg0, ss0), (vr1, sg1, ss1))

        def issue(t, b):
            vr, sg, _ = bufs[b]
            pltpu.async_copy(vcat_hbm.at[srcg.at[t]], vr, sg)

        def wait(t, b):
            vr, sg, _ = bufs[b]
            pltpu.make_async_copy(vcat_hbm.at[srcg.at[t]], vr, sg).wait()

        def wait_sct(b):
            vr, _, ss = bufs[b]
            pltpu.make_async_copy(zagg_hbm.at[pl.ds(0, CH)], vr, ss).wait()

        def scale_scatter(t, b):
            vr, _, ss = bufs[b]
            for grp in range(CH // 16):
                sl = pl.ds(grp * 16, 16)
                d16 = dstg[t, sl]
                rd = plsc.load_gather(rdenom, [d16])
                wbuf[sl] = exg[pl.ds(t * CH + grp * 16, 16)] * rd

            def ebody(e, carry3):
                wsp = plsc.load_gather(wbuf, [jnp.full((16,), e, jnp.int32)])
                for cb in range(HH // 16):
                    slc = pl.ds(cb * 16, 16)
                    vr[e, slc] = vr[e, slc] * wsp
                return carry3

            lax.fori_loop(0, CH, ebody, jnp.int32(0), unroll=2)
            pltpu.async_copy(vr, agg_sp.at[dstg.at[t]], ss, add=True)

        def group_body(gi, carry):
            pltpu.sync_copy(dst_hbm.at[s, pl.ds(gi * GC, GC)], dstg)
            pltpu.sync_copy(src_hbm.at[s, pl.ds(gi * GC, GC)], srcg)
            pltpu.sync_copy(ex_hbm.at[s, pl.ds(gi * GC * CH, GC * CH)], exg)

            def offbody(t, carry2):
                for j in range(CH // 16):
                    sl = pl.ds(j * 16, 16)
                    srcg[t, sl] = srcg[t, sl] + srcoff
                return carry2

            lax.fori_loop(0, GC, offbody, jnp.int32(0))
            issue(0, 0)

            def pair_body(p, carry2):
                t0 = p * 2
                t1 = t0 + 1
                wait(t0, 0)
                scale_scatter(t0, 0)
                wait(t1, 1)
                scale_scatter(t1, 1)

                @pl.when(t0 + 2 < GC)
                def _():
                    wait_sct(0)
                    issue(t0 + 2, 0)

                @pl.when(t1 + 2 < GC)
                def _():
                    wait_sct(1)
                    issue(t1 + 2, 1)

                return carry2

            issue(1, 1)
            lax.fori_loop(0, GC // 2, pair_body, jnp.int32(0))
            wait_sct(0)
            wait_sct(1)
            return carry

        lax.fori_loop(0, NG, group_body, jnp.int32(0))
        plsc.subcore_barrier()
        m8 = [jnp.full((16,), NEG_BIG, jnp.float32) for _ in range(HH // 16)]
        for blk in range(SLICE // CH):
            base = s * SLICE + blk * CH
            pltpu.sync_copy(agg_sp.at[pl.ds(base, CH)], vr1)
            pltpu.sync_copy(sxc_hbm.at[pl.ds(c * NP + base, CH)], vr0)

            def pbody(r, carry):
                row_ok = base + r < N
                out = []
                for j in range(HH // 16):
                    slj = pl.ds(j * 16, 16)
                    hj = vr1[r, slj] + vr0[r, slj]
                    hj = jnp.where(row_ok, hj, jnp.full((16,), NEG_BIG, jnp.float32))
                    out.append(jnp.maximum(carry[j], hj))
                return tuple(out)

            m8 = lax.fori_loop(0, CH, pbody, tuple(m8))
            m8 = list(m8)
        for j in range(HH // 16):
            wbuf[pl.ds(j * 16, 16)] = m8[j]
        pltpu.sync_copy(wbuf, pout_hbm.at[c, s])

    return k3


def _build_mlp1(B, G, H, P, M):
    TK = 1000
    nk = G // TK
    grid = (nk,)

    def body(ct_ref, wm1_ref, pert_ref, wp_ref, bp_ref, w1b_ref, w1c_ref,
             pooled_ref, bm1_ref, out_ref):
        i = pl.program_id(0)

        @pl.when(i == 0)
        def _():
            out_ref[...] = jnp.zeros_like(out_ref)

        out_ref[...] += jax.lax.dot_general(
            ct_ref[...], wm1_ref[...], (((0,), (0,)), ((), ())),
            preferred_element_type=jnp.float32)

        @pl.when(i == nk - 1)
        def _():
            emb = jnp.dot(pert_ref[...], wp_ref[...],
                          preferred_element_type=jnp.float32) + bp_ref[...]
            acc2 = jnp.dot(emb, w1b_ref[...], preferred_element_type=jnp.float32)
            t = jnp.dot(pooled_ref[...], w1c_ref[...],
                        preferred_element_type=jnp.float32)
            z = out_ref[...] + acc2 + t + bm1_ref[...]
            out_ref[...] = jax.nn.softplus(z)

    return pl.pallas_call(
        body,
        grid=grid,
        in_specs=[
            pl.BlockSpec((TK, B), lambda i: (i, 0)),
            pl.BlockSpec((TK, M), lambda i: (i, 0)),
            pl.BlockSpec((B, P), lambda i: (0, 0)),
            pl.BlockSpec((P, P), lambda i: (0, 0)),
            pl.BlockSpec((1, P), lambda i: (0, 0)),
            pl.BlockSpec((P, M), lambda i: (0, 0)),
            pl.BlockSpec((H, M), lambda i: (0, 0)),
            pl.BlockSpec((1, H), lambda i: (0, 0)),
            pl.BlockSpec((1, M), lambda i: (0, 0)),
        ],
        out_specs=pl.BlockSpec((B, M), lambda i: (0, 0)),
        out_shape=jax.ShapeDtypeStruct((B, M), jnp.float32),
    )


def _build_mlp2(B, G, M):
    TKM = 256
    nk = M // TKM
    grid = (nk,)

    def body(h1_ref, w2_ref, b2_ref, out_ref):
        i = pl.program_id(0)

        @pl.when(i == 0)
        def _():
            out_ref[...] = jnp.zeros_like(out_ref)

        out_ref[...] += jnp.dot(h1_ref[...], w2_ref[...],
                                preferred_element_type=jnp.float32)

        @pl.when(i == nk - 1)
        def _():
            out_ref[...] += b2_ref[...]

    return pl.pallas_call(
        body,
        grid=grid,
        in_specs=[
            pl.BlockSpec((B, TKM), lambda i: (0, i)),
            pl.BlockSpec((TKM, G), lambda i: (i, 0)),
            pl.BlockSpec((1, G), lambda i: (0, 0)),
        ],
        out_specs=pl.BlockSpec((B, G), lambda i: (0, 0)),
        out_shape=jax.ShapeDtypeStruct((B, G), jnp.float32),
    )


def kernel(x, edge_index, ctrl, pert, pos, Wq, bq, Wk, bk, Wv, bv,
           Wskip, bskip, W1, b1, Wp, bp, Wm1, bm1, Wm2, bm2):
    N, D = x.shape
    E = edge_index.shape[1]
    H = Wq.shape[1]
    B, G = ctrl.shape
    P = pert.shape[1]
    M = Wm1.shape[1]
    HH = H // 2
    NP = ((N + NW * 16 - 1) // (NW * 16)) * (NW * 16)
    ND = NP
    SLICE = ND // NSUB
    Ep = ((E + NW * CH - 1) // (NW * CH)) * (NW * CH)
    EPW = Ep // NW
    CHW = EPW // CH
    ESUB = Ep // NSUB
    CHS = ESUB // CH

    xp = jnp.pad(x, ((0, NP - N), (0, 0)))
    pad_ids = N + (jnp.arange(Ep - E, dtype=jnp.int32) % (NP - N))
    src = jnp.concatenate([edge_index[0], pad_ids])
    dst = jnp.concatenate([edge_index[1], pad_ids])
    CH1 = 64
    CHW1 = EPW // CH1
    dstw = dst.reshape(NW, CHW, CH)
    srcw = src.reshape(NW, CHW, CH)
    dstw1 = dst.reshape(NW, CHW1, CH1)
    srcw1 = src.reshape(NW, CHW1, CH1)
    dsts = dst.reshape(NSUB, CHS, CH)
    srcs = src.reshape(NSUB, CHS, CH)

    wbig = jnp.concatenate([Wq, Wk, Wv, Wskip + W1], axis=1)
    bbig = jnp.concatenate([bq, bk, bv, bskip + b1])[None, :]
    qk4, v0, v1, sxc = _build_proj(NP, D, H)(xp, wbig, bbig)
    sxcat = sxc.reshape(2 * NP, HH)
    qkflat = qk4.reshape(2 * NP, H)

    alpha, mx = _build_k1(NP, Ep, H, CH1, CHW1, EPW)(qkflat, dstw1, srcw1)
    znd = jnp.zeros((ND,), jnp.float32)
    ex, den2 = _build_k2(ND, SLICE, CHW, EPW)(alpha, dstw, mx, znd)
    vcat = jnp.concatenate([v0, v1], axis=0)
    zagg = jnp.zeros((SLICE, HH), jnp.float32)
    exs = ex.reshape(NSUB, ESUB)
    pout = _build_k3(N, NP, ND, SLICE, HH, CHS, ESUB)(
        vcat, exs, dsts, srcs, den2, zagg, sxcat)

    pooled = jnp.max(pout, axis=1).reshape(1, H)

    ctrl_t = ctrl.T
    w1c = Wm1[G:G + H]
    w1b = Wm1[G + H:]
    h1 = _build_mlp1(B, G, H, P, M)(ctrl_t, Wm1, pert, Wp, bp[None], w1b,
                                    w1c, pooled, bm1[None])
    out = _build_mlp2(B, G, M)(h1, Wm2, bm2[None])
    return out


# speedup vs baseline: 1.8568x; 1.0058x over previous
import functools

import jax
import jax.numpy as jnp
from jax import lax
from jax.experimental import pallas as pl
from jax.experimental.pallas import tpu as pltpu
from jax.experimental.pallas import tpu_sc as plsc

NEG_BIG = -3.0e38
_SC_PARAMS = pltpu.CompilerParams(use_tc_tiling_on_sc=False,
                                  needs_layout_passes=False)
CH = 128
NW = 32
NSUB = 16


def _tree_sum(vs):
    vs = list(vs)
    while len(vs) > 1:
        nxt = [vs[i] + vs[i + 1] for i in range(0, len(vs) - 1, 2)]
        if len(vs) % 2:
            nxt.append(vs[-1])
        vs = nxt
    return vs[0]


def _build_proj(NP, D, H):
    TN = 512
    grid = (NP // TN,)

    def body(x_ref, w_ref, b_ref, qk_ref, v_ref, s_ref):
        res = jnp.dot(x_ref[...], w_ref[...],
                      preferred_element_type=jnp.float32) + b_ref[...]
        qk_ref[0] = res[:, 0:H]
        qk_ref[1] = res[:, H:2 * H]
        v_ref[0] = res[:, 2 * H:2 * H + H // 2]
        v_ref[1] = res[:, 2 * H + H // 2:3 * H]
        s_ref[0] = res[:, 3 * H:3 * H + H // 2]
        s_ref[1] = res[:, 3 * H + H // 2:4 * H]

    return pl.pallas_call(
        body,
        grid=grid,
        in_specs=[
            pl.BlockSpec((TN, D), lambda i: (i, 0)),
            pl.BlockSpec((D, 4 * H), lambda i: (0, 0)),
            pl.BlockSpec((1, 4 * H), lambda i: (0, 0)),
        ],
        out_specs=[
            pl.BlockSpec((2, TN, H), lambda i: (0, i, 0)),
            pl.BlockSpec((2, TN, H // 2), lambda i: (0, i, 0)),
            pl.BlockSpec((2, TN, H // 2), lambda i: (0, i, 0)),
        ],
        out_shape=[
            jax.ShapeDtypeStruct((2, NP, H), jnp.float32),
            jax.ShapeDtypeStruct((2, NP, H // 2), jnp.float32),
            jax.ShapeDtypeStruct((2, NP, H // 2), jnp.float32),
        ],
    )


def _build_k1(NP, Ep, H, CH1, CHW1, EPW):
    inv_sqrt_h = 1.0 / (H ** 0.5)
    mesh = plsc.VectorSubcoreMesh(core_axis_name="c", subcore_axis_name="s")

    @functools.partial(
        pl.kernel,
        out_type=(jax.ShapeDtypeStruct((NW, EPW), jnp.float32),
                  jax.ShapeDtypeStruct((NW, 16), jnp.float32)),
        mesh=mesh,
        compiler_params=_SC_PARAMS,
        scratch_types=[
            pltpu.VMEM((CHW1, CH1), jnp.int32),
            pltpu.VMEM((CHW1, CH1), jnp.int32),
            pltpu.VMEM((CH1, H), jnp.float32),
            pltpu.VMEM((CH1, H), jnp.float32),
            pltpu.VMEM((CH1, H), jnp.float32),
            pltpu.VMEM((CH1, H), jnp.float32),
            pltpu.VMEM((EPW,), jnp.float32),
            pltpu.VMEM((16, 16), jnp.float32),
            pltpu.VMEM((16,), jnp.float32),
            pltpu.SemaphoreType.DMA,
            pltpu.SemaphoreType.DMA,
            pltpu.SemaphoreType.DMA,
            pltpu.SemaphoreType.DMA,
        ],
    )
    def k1(qk_hbm, dst_hbm, src_hbm, alpha_hbm, mx_hbm,
           dst2d, src2d, qr0, kr0, qr1, kr1, alphabig, tbuf, mxbuf,
           sq0, sk0, sq1, sk1):
        c = lax.axis_index("c")
        s = lax.axis_index("s")
        wid = s * 2 + c
        pltpu.sync_copy(dst_hbm.at[wid], dst2d)
        pltpu.sync_copy(src_hbm.at[wid], src2d)
        iota = jnp.arange(16, dtype=jnp.int32)
        inv = jnp.float32(inv_sqrt_h)
        koff = NP

        def offbody(t, carry):
            for j in range(CH1 // 16):
                sl = pl.ds(j * 16, 16)
                src2d[t, sl] = src2d[t, sl] + koff
            return carry

        lax.fori_loop(0, CHW1, offbody, jnp.int32(0))
        bufs = ((qr0, kr0, sq0, sk0), (qr1, kr1, sq1, sk1))

        def issue(t, b):
            qr, kr, sq, sk = bufs[b]
            pltpu.async_copy(qk_hbm.at[dst2d.at[t]], qr, sq)
            pltpu.async_copy(qk_hbm.at[src2d.at[t]], kr, sk)

        def wait(t, b):
            qr, kr, sq, sk = bufs[b]
            pltpu.make_async_copy(qk_hbm.at[dst2d.at[t]], qr, sq).wait()
            pltpu.make_async_copy(qk_hbm.at[src2d.at[t]], kr, sk).wait()

        def compute(t, b, mxv):
            qr, kr, _, _ = bufs[b]
            for grp in range(CH1 // 16):

                def ebody(e, carry):
                    r = grp * 16 + e
                    ps = [qr[r, pl.ds(j * 16, 16)] * kr[r, pl.ds(j * 16, 16)]
                          for j in range(H // 16)]
                    tbuf[e, pl.ds(0, 16)] = _tree_sum(ps)
                    return carry

                lax.fori_loop(0, 16, ebody, jnp.int32(0))
                cols = [plsc.load_gather(tbuf, [iota, jnp.full((16,), j, jnp.int32)])
                        for j in range(16)]
                a16 = _tree_sum(cols) * inv
                alphabig[pl.ds(t * CH1 + grp * 16, 16)] = a16
                mxv = jnp.maximum(mxv, a16)
            return mxv

        issue(0, 0)

        def pair_body(p, mxv):
            t0 = p * 2
            t1 = t0 + 1
            issue(t1, 1)
            wait(t0, 0)
            mxv = compute(t0, 0, mxv)

            @pl.when(t0 + 2 < CHW1)
            def _():
                issue(t0 + 2, 0)

            wait(t1, 1)
            mxv = compute(t1, 1, mxv)
            return mxv

        mxv = lax.fori_loop(0, CHW1 // 2, pair_body,
                            jnp.full((16,), NEG_BIG, jnp.float32))
        mxbuf[...] = mxv
        pltpu.sync_copy(alphabig, alpha_hbm.at[wid])
        pltpu.sync_copy(mxbuf, mx_hbm.at[wid])

    return k1


def _build_k2(ND, SLICE, CHW, EPW):
    mesh = plsc.VectorSubcoreMesh(core_axis_name="c", subcore_axis_name="s")

    @functools.partial(
        pl.kernel,
        out_type=(jax.ShapeDtypeStruct((NW, EPW), jnp.float32),
                  jax.ShapeDtypeStruct((2, ND), jnp.float32)),
        mesh=mesh,
        compiler_params=_SC_PARAMS,
        scratch_types=[
            pltpu.VMEM((NW, 16), jnp.float32),
            pltpu.VMEM((CHW, CH), jnp.int32),
            pltpu.VMEM((EPW,), jnp.float32),
            pltpu.VMEM((EPW,), jnp.float32),
            pltpu.VMEM_SHARED((ND,), jnp.float32),
        ],
    )
    def k2(alpha_hbm, dst_hbm, mx_hbm, znd_hbm, ex_hbm, den_hbm,
           mxbuf, dst2d, alphabig, exbig, denom_sp):
        c = lax.axis_index("c")
        s = lax.axis_index("s")
        wid = s * 2 + c
        pltpu.sync_copy(mx_hbm, mxbuf)

        def mbody(i, m):
            return jnp.maximum(m, mxbuf[i])

        m = lax.fori_loop(0, NW, mbody, jnp.full((16,), NEG_BIG, jnp.float32))
        cmax = jnp.max(m)
        cvec = jnp.full((16,), cmax)
        pltpu.sync_copy(znd_hbm.at[pl.ds(s * SLICE, SLICE)],
                        denom_sp.at[pl.ds(s * SLICE, SLICE)])
        pltpu.sync_copy(alpha_hbm.at[wid], alphabig)
        pltpu.sync_copy(dst_hbm.at[wid], dst2d)

        def gbody(g, carry):
            sl = pl.ds(g * 16, 16)
            exbig[sl] = jnp.exp(alphabig[sl] - cvec)
            return carry

        lax.fori_loop(0, EPW // 16, gbody, jnp.int32(0))
        pltpu.sync_copy(exbig, ex_hbm.at[wid])
        plsc.subcore_barrier()

        def sbody(t, carry):
            pltpu.sync_copy(exbig.at[pl.ds(t * CH, CH)],
                            denom_sp.at[dst2d.at[t]], add=True)
            return carry

        lax.fori_loop(0, CHW, sbody, jnp.int32(0))
        plsc.subcore_barrier()
        pltpu.sync_copy(denom_sp.at[pl.ds(s * SLICE, SLICE)],
                        den_hbm.at[c, pl.ds(s * SLICE, SLICE)])

    return k2


def _build_k3(N, NP, ND, SLICE, HH, CHS, ESUB):
    last_rows = N - (NSUB - 1) * SLICE
    GC = 8
    NG = CHS // GC
    DB = ND // 4
    mesh = plsc.VectorSubcoreMesh(core_axis_name="c", subcore_axis_name="s")

    @functools.partial(
        pl.kernel,
        out_type=jax.ShapeDtypeStruct((2, NSUB, HH), jnp.float32),
        mesh=mesh,
        compiler_params=_SC_PARAMS,
        scratch_types=[
            pltpu.VMEM((ND,), jnp.float32),
            pltpu.VMEM((DB,), jnp.float32),
            pltpu.VMEM((GC, CH), jnp.int32),
            pltpu.VMEM((GC, CH), jnp.int32),
            pltpu.VMEM((GC * CH,), jnp.float32),
            pltpu.VMEM((CH,), jnp.float32),
            pltpu.VMEM((CH, HH), jnp.float32),
            pltpu.VMEM((CH, HH), jnp.float32),
            pltpu.VMEM_SHARED((ND, HH), jnp.float32),
            pltpu.SemaphoreType.DMA,
            pltpu.SemaphoreType.DMA,
            pltpu.SemaphoreType.DMA,
            pltpu.SemaphoreType.DMA,
        ],
    )
    def k3(vcat_hbm, ex_hbm, dst_hbm, src_hbm, den_hbm, zagg_hbm, sxc_hbm,
           pout_hbm,
           rdenom, dbuf, dstg, srcg, exg, wbuf, vr0, vr1, agg_sp,
           sg0, sg1, ss0, ss1):
        c = lax.axis_index("c")
        s = lax.axis_index("s")
        srcoff = c * NP
        pltpu.sync_copy(den_hbm.at[0], rdenom)
        for blk in range(4):
            pltpu.sync_copy(den_hbm.at[1, pl.ds(blk * DB, DB)], dbuf)

            def rbody(i, carry, _blk=blk):
                sl16 = pl.ds(_blk * DB + i * 16, 16)
                rdenom[sl16] = 1.0 / (rdenom[sl16] + dbuf[pl.ds(i * 16, 16)]
                                      + jnp.float32(1e-16))
                return carry

            lax.fori_loop(0, DB // 16, rbody, jnp.int32(0))
        pltpu.sync_copy(zagg_hbm, agg_sp.at[pl.ds(s * SLICE, SLICE)])
        plsc.subcore_barrier()
        bufs = ((vr0, sg0, ss0), (vr1, sg1, ss1))

        def issue(t, b):
            vr, sg, _ = bufs[b]
            pltpu.async_copy(vcat_hbm.at[srcg.at[t]], vr, sg)

        def wait(t, b):
            vr, sg, _ = bufs[b]
            pltpu.make_async_copy(vcat_hbm.at[srcg.at[t]], vr, sg).wait()

        def wait_sct(b):
            vr, _, ss = bufs[b]
            pltpu.make_async_copy(zagg_hbm.at[pl.ds(0, CH)], vr, ss).wait()

        def scale_scatter(t, b):
            vr, _, ss = bufs[b]
            for grp in range(CH // 16):
                sl = pl.ds(grp * 16, 16)
                d16 = dstg[t, sl]
                rd = plsc.load_gather(rdenom, [d16])
                wbuf[sl] = exg[pl.ds(t * CH + grp * 16, 16)] * rd

            def ebody(e, carry3):
                wsp = plsc.load_gather(wbuf, [jnp.full((16,), e, jnp.int32)])
                for cb in range(HH // 16):
                    slc = pl.ds(cb * 16, 16)
                    vr[e, slc] = vr[e, slc] * wsp
                return carry3

            lax.fori_loop(0, CH, ebody, jnp.int32(0), unroll=2)
            pltpu.async_copy(vr, agg_sp.at[dstg.at[t]], ss, add=True)

        def group_body(gi, carry):
            pltpu.sync_copy(dst_hbm.at[s, pl.ds(gi * GC, GC)], dstg)
            pltpu.sync_copy(src_hbm.at[s, pl.ds(gi * GC, GC)], srcg)
            pltpu.sync_copy(ex_hbm.at[s, pl.ds(gi * GC * CH, GC * CH)], exg)

            def offbody(t, carry2):
                for j in range(CH // 16):
                    sl = pl.ds(j * 16, 16)
                    srcg[t, sl] = srcg[t, sl] + srcoff
                return carry2

            lax.fori_loop(0, GC, offbody, jnp.int32(0))
            issue(0, 0)

            def pair_body(p, carry2):
                t0 = p * 2
                t1 = t0 + 1
                wait(t0, 0)
                scale_scatter(t0, 0)
                wait(t1, 1)
                scale_scatter(t1, 1)

                @pl.when(t0 + 2 < GC)
                def _():
                    wait_sct(0)
                    issue(t0 + 2, 0)

                @pl.when(t1 + 2 < GC)
                def _():
                    wait_sct(1)
                    issue(t1 + 2, 1)

                return carry2

            issue(1, 1)
            lax.fori_loop(0, GC // 2, pair_body, jnp.int32(0))
            wait_sct(0)
            wait_sct(1)
            return carry

        lax.fori_loop(0, NG, group_body, jnp.int32(0))
        plsc.subcore_barrier()
        m8 = [jnp.full((16,), NEG_BIG, jnp.float32) for _ in range(HH // 16)]
        for blk in range(SLICE // CH):
            base = s * SLICE + blk * CH
            pltpu.sync_copy(agg_sp.at[pl.ds(base, CH)], vr1)
            pltpu.sync_copy(sxc_hbm.at[pl.ds(c * NP + base, CH)], vr0)

            def pbody(r, carry):
                row_ok = base + r < N
                out = []
                for j in range(HH // 16):
                    slj = pl.ds(j * 16, 16)
                    hj = vr1[r, slj] + vr0[r, slj]
                    hj = jnp.where(row_ok, hj, jnp.full((16,), NEG_BIG, jnp.float32))
                    out.append(jnp.maximum(carry[j], hj))
                return tuple(out)

            m8 = lax.fori_loop(0, CH, pbody, tuple(m8))
            m8 = list(m8)
        for j in range(HH // 16):
            wbuf[pl.ds(j * 16, 16)] = m8[j]
        pltpu.sync_copy(wbuf, pout_hbm.at[c, s])

    return k3


def _build_mlp1(B, G, H, P, M):
    TK = 1000
    nk = G // TK
    grid = (nk,)

    def body(ct_ref, wm1_ref, pert_ref, wp_ref, bp_ref, w1b_ref, w1c_ref,
             pooled_ref, bm1_ref, out_ref):
        i = pl.program_id(0)

        @pl.when(i == 0)
        def _():
            out_ref[...] = jnp.zeros_like(out_ref)

        out_ref[...] += jax.lax.dot_general(
            ct_ref[...], wm1_ref[...], (((0,), (0,)), ((), ())),
            preferred_element_type=jnp.float32)

        @pl.when(i == nk - 1)
        def _():
            emb = jnp.dot(pert_ref[...], wp_ref[...],
                          preferred_element_type=jnp.float32) + bp_ref[...]
            acc2 = jnp.dot(emb, w1b_ref[...], preferred_element_type=jnp.float32)
            t = jnp.dot(pooled_ref[...], w1c_ref[...],
                        preferred_element_type=jnp.float32)
            z = out_ref[...] + acc2 + t + bm1_ref[...]
            out_ref[...] = jax.nn.softplus(z)

    return pl.pallas_call(
        body,
        grid=grid,
        in_specs=[
            pl.BlockSpec((TK, B), lambda i: (i, 0)),
            pl.BlockSpec((TK, M), lambda i: (i, 0)),
            pl.BlockSpec((B, P), lambda i: (0, 0)),
            pl.BlockSpec((P, P), lambda i: (0, 0)),
            pl.BlockSpec((1, P), lambda i: (0, 0)),
            pl.BlockSpec((P, M), lambda i: (0, 0)),
            pl.BlockSpec((H, M), lambda i: (0, 0)),
            pl.BlockSpec((1, H), lambda i: (0, 0)),
            pl.BlockSpec((1, M), lambda i: (0, 0)),
        ],
        out_specs=pl.BlockSpec((B, M), lambda i: (0, 0)),
        out_shape=jax.ShapeDtypeStruct((B, M), jnp.float32),
    )


def _build_mlp2(B, G, M):
    TKM = 256
    nk = M // TKM
    grid = (nk,)

    def body(h1_ref, w2_ref, b2_ref, out_ref):
        i = pl.program_id(0)

        @pl.when(i == 0)
        def _():
            out_ref[...] = jnp.zeros_like(out_ref)

        out_ref[...] += jnp.dot(h1_ref[...], w2_ref[...],
                                preferred_element_type=jnp.float32)

        @pl.when(i == nk - 1)
        def _():
            out_ref[...] += b2_ref[...]

    return pl.pallas_call(
        body,
        grid=grid,
        in_specs=[
            pl.BlockSpec((B, TKM), lambda i: (0, i)),
            pl.BlockSpec((TKM, G), lambda i: (i, 0)),
            pl.BlockSpec((1, G), lambda i: (0, 0)),
        ],
        out_specs=pl.BlockSpec((B, G), lambda i: (0, 0)),
        out_shape=jax.ShapeDtypeStruct((B, G), jnp.float32),
    )


def kernel(x, edge_index, ctrl, pert, pos, Wq, bq, Wk, bk, Wv, bv,
           Wskip, bskip, W1, b1, Wp, bp, Wm1, bm1, Wm2, bm2):
    N, D = x.shape
    E = edge_index.shape[1]
    H = Wq.shape[1]
    B, G = ctrl.shape
    P = pert.shape[1]
    M = Wm1.shape[1]
    HH = H // 2
    NP = ((N + NW * 16 - 1) // (NW * 16)) * (NW * 16)
    ND = NP
    SLICE = ND // NSUB
    Ep = ((E + NW * CH - 1) // (NW * CH)) * (NW * CH)
    EPW = Ep // NW
    CHW = EPW // CH
    ESUB = Ep // NSUB
    CHS = ESUB // CH

    xp = jnp.pad(x, ((0, NP - N), (0, 0)))
    pad_ids = N + (jnp.arange(Ep - E, dtype=jnp.int32) % (NP - N))
    src = jnp.concatenate([edge_index[0], pad_ids])
    dst = jnp.concatenate([edge_index[1], pad_ids])
    CH1 = 64
    CHW1 = EPW // CH1
    dstw = dst.reshape(NW, CHW, CH)
    srcw = src.reshape(NW, CHW, CH)
    dstw1 = dst.reshape(NW, CHW1, CH1)
    srcw1 = src.reshape(NW, CHW1, CH1)
    dsts = dst.reshape(NSUB, CHS, CH)
    srcs = src.reshape(NSUB, CHS, CH)

    wbig = jnp.concatenate([Wq, Wk, Wv, Wskip + W1], axis=1)
    bbig = jnp.concatenate([bq, bk, bv, bskip + b1])[None, :]
    qk4, vc, sxc = _build_proj(NP, D, H)(xp, wbig, bbig)
    sxcat = sxc.reshape(2 * NP, HH)
    vcat = vc.reshape(2 * NP, HH)
    qkflat = qk4.reshape(2 * NP, H)

    alpha, mx = _build_k1(NP, Ep, H, CH1, CHW1, EPW)(qkflat, dstw1, srcw1)
    znd = jnp.zeros((ND,), jnp.float32)
    ex, den2 = _build_k2(ND, SLICE, CHW, EPW)(alpha, dstw, mx, znd)
    zagg = jnp.zeros((SLICE, HH), jnp.float32)
    exs = ex.reshape(NSUB, ESUB)
    pout = _build_k3(N, NP, ND, SLICE, HH, CHS, ESUB)(
        vcat, exs, dsts, srcs, den2, zagg, sxcat)

    pooled = jnp.max(pout, axis=1).reshape(1, H)

    ctrl_t = ctrl.T
    w1c = Wm1[G:G + H]
    w1b = Wm1[G + H:]
    h1 = _build_mlp1(B, G, H, P, M)(ctrl_t, Wm1, pert, Wp, bp[None], w1b,
                                    w1c, pooled, bm1[None])
    out = _build_mlp2(B, G, M)(h1, Wm2, bm2[None])
    return out
